# baseline (speedup 1.0000x reference)
import functools

import jax
import jax.numpy as jnp
from jax import lax
from jax.experimental import pallas as pl
from jax.experimental.pallas import tpu as pltpu
from jax.experimental.pallas import tpu_sc as plsc

N = 10000
D = 128
L = 16
NT = 32
PAD = 640

_f32 = jnp.float32
_i32 = jnp.int32


def _wid():
    return lax.axis_index("s") * 2 + lax.axis_index("c")


def _sread(ref, idx):
    return ref[pl.ds(idx, L)][0]


def _clamp_offsets(idxbuf, offbuf, base_val, limit):
    for j in range(128 // L):
        off = idxbuf[pl.ds(L * j, L)] - base_val
        ok = (off >= 0) & (off < limit)
        offbuf[pl.ds(L * j, L)] = jnp.where(ok, off, limit)


def _row_add(dst, dst_row, src, src_row):
    for j in range(D // L):
        v = src[src_row, pl.ds(L * j, L)]
        plsc.addupdate(dst.at[dst_row, pl.ds(L * j, L)], v)


def _zero_rows(buf, nrows):
    z = jnp.zeros((L,), _f32)

    def body(r, _):
        for j in range(D // L):
            buf[r, pl.ds(L * j, L)] = z
        return 0

    lax.fori_loop(0, nrows, body, 0)


def _mm_body(x_ref, w_ref, o_ref):
    o_ref[...] = jnp.dot(x_ref[...], w_ref[...], preferred_element_type=_f32)


def _tc_matmul(x, w):
    nnz = x.shape[0]
    br = 2048
    g = (nnz + br - 1) // br
    return pl.pallas_call(
        _mm_body,
        grid=(g,),
        in_specs=[
            pl.BlockSpec((br, D), lambda i: (i, 0)),
            pl.BlockSpec((D, D), lambda i: (0, 0)),
        ],
        out_specs=pl.BlockSpec((br, D), lambda i: (i, 0)),
        out_shape=jax.ShapeDtypeStruct((nnz, D), _f32),
    )(x, w)


def _node_mm_body(d_ref, p1_ref, p0_ref, x5_ref, w2, w3, w4, w5, w6, b_ref,
                  g0_ref, g1_ref):
    dd = d_ref[...]
    g0_ref[...] = (jnp.dot(dd, w2[...], preferred_element_type=_f32)
                   + jnp.dot(p0_ref[...], w5[...], preferred_element_type=_f32)
                   + jnp.dot(x5_ref[...], w6[...], preferred_element_type=_f32)
                   + b_ref[...])
    g1_ref[...] = (jnp.dot(dd, w3[...], preferred_element_type=_f32)
                   + jnp.dot(p1_ref[...], w4[...], preferred_element_type=_f32))


def _tc_node_matmul(diag, pool1, pool0, x5, w2, w3, w4, w5, w6, b2d):
    br = 1000
    g = N // br
    full = pl.BlockSpec((D, D), lambda i: (0, 0))
    blk = pl.BlockSpec((br, D), lambda i: (i, 0))
    return pl.pallas_call(
        _node_mm_body,
        grid=(g,),
        in_specs=[blk, blk, blk, blk, full, full, full, full, full,
                  pl.BlockSpec((1, D), lambda i: (0, 0))],
        out_specs=[blk, blk],
        out_shape=[jax.ShapeDtypeStruct((N, D), _f32),
                   jax.ShapeDtypeStruct((N, D), _f32)],
    )(diag, pool1, pool0, x5, w2, w3, w4, w5, w6, b2d)


def _make_pools_kernel(nnz):
    mesh = plsc.VectorSubcoreMesh(core_axis_name="c", subcore_axis_name="s")
    SN = 120
    NS = 10016
    ZR = 624

    @functools.partial(
        pl.kernel,
        out_type=[jax.ShapeDtypeStruct((N, D), _f32),
                  jax.ShapeDtypeStruct((2, N, D), _f32),
                  jax.ShapeDtypeStruct((N, D), _f32)],
        mesh=mesh,
        scratch_types=[
            pltpu.VMEM((SN + 1, D), _f32),
            pltpu.VMEM((SN + 1, D), _f32),
            pltpu.VMEM((128, D), _f32),
            pltpu.VMEM((144,), _i32),
            pltpu.VMEM((144,), _i32),
            pltpu.VMEM((128,), _i32),
            pltpu.VMEM((128,), _i32),
            pltpu.VMEM((144,), _i32),
            pltpu.VMEM((144,), _i32),
            pltpu.VMEM((112,), _i32),
            pltpu.VMEM_SHARED((NS, D), _f32),
        ],
    )
    def pools(x_hbm, i0t_hbm, i1t_hbm, xbs_hbm,
              p1_hbm, p0p_hbm, dg_hbm,
              p1buf, dbuf, xbuf, i0b, i1b, i1c, idb, offb, dfb, xbsb, spool):
        cid = lax.axis_index("c")
        sid = lax.axis_index("s")
        wid = sid * 2 + cid

        pltpu.sync_copy(xbs_hbm, xbsb)

        _zero_rows(xbuf, 128)
        z0 = pl.multiple_of(sid * ZR, 8)
        for h in range(4):
            pltpu.sync_copy(xbuf.at[pl.ds(0, 128)],
                            spool.at[pl.ds(z0 + 128 * h, 128)])

        @pl.when(sid < 15)
        def _():
            pltpu.sync_copy(xbuf.at[pl.ds(0, 112)],
                            spool.at[pl.ds(z0 + 512, 112)])

        @pl.when(sid == 15)
        def _():
            pltpu.sync_copy(xbuf.at[pl.ds(0, 128)],
                            spool.at[pl.ds(z0 + 512, 128)])

        plsc.subcore_barrier()

        def chunk_body(ci, _):
            n0 = pl.multiple_of(wid * (3 * SN) + ci * SN, 8)
            r0 = _sread(xbsb, 3 * wid + ci)
            r1 = _sread(xbsb, 3 * wid + ci + 1)
            base = r0 & ~7
            nb = (r1 - base + 127) // 128

            _zero_rows(p1buf, SN + 1)
            _zero_rows(dbuf, SN + 1)

            def batch_body(bi, _):
                s = pl.multiple_of(base + bi * 128, 8)
                pltpu.sync_copy(i0t_hbm.at[pl.ds(s, 128)], i0b.at[pl.ds(0, 128)])
                pltpu.sync_copy(i1t_hbm.at[pl.ds(s, 128)], i1b.at[pl.ds(0, 128)])
                for j in range(128 // L):
                    v = s + L * j + lax.iota(_i32, L)
                    idb[pl.ds(L * j, L)] = jnp.minimum(v, nnz - 1)
                pltpu.sync_copy(x_hbm.at[idb], xbuf)
                for j in range(128 // L):
                    pos = s + L * j + lax.iota(_i32, L)
                    okp = (pos >= r0) & (pos < r1)
                    i1v = i1b[pl.ds(L * j, L)]
                    i1c[pl.ds(L * j, L)] = jnp.where(okp, i1v, N)
                pltpu.sync_copy(xbuf, spool.at[i1c], add=True)

                _clamp_offsets(i0b, offb, n0, SN)
                for j in range(128 // L):
                    offv = offb[pl.ds(L * j, L)]
                    eq = i0b[pl.ds(L * j, L)] == i1b[pl.ds(L * j, L)]
                    dfb[pl.ds(L * j, L)] = jnp.where(eq, offv, SN)

                def row_body(g, _):
                    for u in range(2):
                        k = 2 * g + u
                        offc = _sread(offb, k)
                        _row_add(p1buf, offc, xbuf, k)
                        dof = _sread(dfb, k)

                        @pl.when(dof < SN)
                        def _():
                            _row_add(dbuf, dof, xbuf, k)

                    return 0

                lax.fori_loop(0, 64, row_body, 0)
                return 0

            lax.fori_loop(0, nb, batch_body, 0)

            full = n0 + SN <= N
            part = n0 == (N // SN) * SN

            @pl.when(full)
            def _():
                pltpu.sync_copy(p1buf.at[pl.ds(0, SN)], p1_hbm.at[pl.ds(n0, SN)])
                pltpu.sync_copy(dbuf.at[pl.ds(0, SN)], dg_hbm.at[pl.ds(n0, SN)])

            @pl.when(part)
            def _():
                rem = N - (N // SN) * SN
                pltpu.sync_copy(p1buf.at[pl.ds(0, rem)], p1_hbm.at[pl.ds(n0, rem)])
                pltpu.sync_copy(dbuf.at[pl.ds(0, rem)], dg_hbm.at[pl.ds(n0, rem)])

            return 0

        lax.fori_loop(0, 3, chunk_body, 0)

        plsc.subcore_barrier()
        w0 = pl.multiple_of(sid * ZR, 8)

        @pl.when(sid < 15)
        def _():
            pltpu.sync_copy(spool.at[pl.ds(w0, ZR)],
                            p0p_hbm.at[cid, pl.ds(w0, ZR)])

        @pl.when(sid == 15)
        def _():
            pltpu.sync_copy(spool.at[pl.ds(w0, 640)],
                            p0p_hbm.at[cid, pl.ds(w0, 640)])

    return pools


def _add_body(a_ref, b_ref, o_ref):
    o_ref[...] = a_ref[...] + b_ref[...]


def _tc_add(a, b):
    br = 1000
    blk = pl.BlockSpec((br, D), lambda i: (i, 0))
    return pl.pallas_call(
        _add_body,
        grid=(N // br,),
        in_specs=[blk, blk],
        out_specs=blk,
        out_shape=jax.ShapeDtypeStruct((N, D), _f32),
    )(a, b)


def _make_x5_kernel():
    mesh = plsc.VectorSubcoreMesh(core_axis_name="c", subcore_axis_name="s")
    SN = 320

    @functools.partial(
        pl.kernel,
        out_type=jax.ShapeDtypeStruct((N, D), _f32),
        mesh=mesh,
        scratch_types=[
            pltpu.VMEM((SN + 1, D), _f32),
            pltpu.VMEM((128, D), _f32),
            pltpu.VMEM((128,), _i32),
            pltpu.VMEM((144,), _i32),
            pltpu.VMEM((144,), _i32),
            pltpu.VMEM((48,), _i32),
        ],
    )
    def x5k(p0_hbm, adst_hbm, asrc_hbm, ap_hbm, x5_hbm,
            x5buf, gbuf, adb, asb, offb, apb):
        wid = _wid()
        n0 = pl.multiple_of(wid * SN, 8)
        pltpu.sync_copy(ap_hbm, apb)
        e0 = _sread(apb, wid)
        e1 = _sread(apb, wid + 1)
        base = e0 & ~7
        nb = (e1 - base + 127) // 128

        _zero_rows(x5buf, SN + 1)

        def batch_body(bi, _):
            s = pl.multiple_of(base + bi * 128, 8)
            pltpu.sync_copy(adst_hbm.at[pl.ds(s, 128)], adb)
            pltpu.sync_copy(asrc_hbm.at[pl.ds(s, 128)], asb.at[pl.ds(0, 128)])
            pltpu.sync_copy(p0_hbm.at[adb], gbuf)
            _clamp_offsets(asb, offb, n0, SN)

            def edge_body(g, _):
                for u in range(2):
                    k = 2 * g + u
                    _row_add(x5buf, _sread(offb, k), gbuf, k)
                return 0

            lax.fori_loop(0, 64, edge_body, 0)
            return 0

        lax.fori_loop(0, nb, batch_body, 0)

        full = n0 + SN <= N
        part = n0 == (N // SN) * SN

        @pl.when(full)
        def _():
            pltpu.sync_copy(x5buf.at[pl.ds(0, SN)], x5_hbm.at[pl.ds(n0, SN)])

        @pl.when(part)
        def _():
            rem = N - (N // SN) * SN
            pltpu.sync_copy(x5buf.at[pl.ds(0, rem)], x5_hbm.at[pl.ds(n0, rem)])

    return x5k


def _make_out_kernel(nnz):
    mesh = plsc.VectorSubcoreMesh(core_axis_name="c", subcore_axis_name="s")
    SR = 256
    nch_total = (nnz + SR - 1) // SR
    last_c = nch_total - 1
    lastv = nnz - last_c * SR
    base_nch = nch_total // NT
    extra = nch_total - base_nch * NT

    @functools.partial(
        pl.kernel,
        out_type=jax.ShapeDtypeStruct((nnz, D), _f32),
        mesh=mesh,
        scratch_types=[
            pltpu.VMEM((SR + 1, D), _f32),
            pltpu.VMEM((SR, D), _f32),
            pltpu.VMEM((128, D), _f32),
            pltpu.VMEM((128,), _i32),
            pltpu.VMEM((128,), _i32),
            pltpu.VMEM((128,), _i32),
            pltpu.VMEM((128,), _i32),
            pltpu.VMEM((128,), _i32),
            pltpu.VMEM((144,), _i32),
            pltpu.VMEM((144,), _i32),
            pltpu.VMEM((688,), _i32),
            pltpu.SemaphoreType.DMA,
            pltpu.SemaphoreType.DMA,
            pltpu.SemaphoreType.DMA,
            pltpu.SemaphoreType.DMA,
        ],
    )
    def outk(g0_hbm, g1_hbm, y1_hbm, i0_hbm, i1_hbm, mpo_hbm, mps_hbm, pp_hbm,
             out_hbm, outbuf, bbuf, ybuf, ib0, ib1, ib2, ib3, pob, psb, offb,
             ppb, sem0, sem1, sem2, sem3):
        wid = _wid()
        nch = base_nch + jnp.where(wid < extra, 1, 0)
        pltpu.sync_copy(pp_hbm, ppb)

        def chunk_body(ci, _):
            c = wid + NT * ci
            ar = pl.multiple_of(c * SR, 8)
            p0 = _sread(ppb, c)
            p1 = _sread(ppb, c + 1)
            base = p0 & ~7
            nb = (p1 - base + 127) // 128

            pltpu.sync_copy(i0_hbm.at[pl.ds(ar, 128)], ib0)
            pltpu.sync_copy(i0_hbm.at[pl.ds(ar + 128, 128)], ib1)
            pltpu.sync_copy(i1_hbm.at[pl.ds(ar, 128)], ib2)
            pltpu.sync_copy(i1_hbm.at[pl.ds(ar + 128, 128)], ib3)
            c0 = pltpu.async_copy(g0_hbm.at[ib0], outbuf.at[pl.ds(0, 128)], sem0)
            c1 = pltpu.async_copy(g0_hbm.at[ib1], outbuf.at[pl.ds(128, 128)], sem1)
            c2 = pltpu.async_copy(g1_hbm.at[ib2], bbuf.at[pl.ds(0, 128)], sem2)
            c3 = pltpu.async_copy(g1_hbm.at[ib3], bbuf.at[pl.ds(128, 128)], sem3)
            c0.wait()
            c1.wait()
            c2.wait()
            c3.wait()

            def init_body(g, _):
                for u in range(2):
                    r = 2 * g + u
                    _row_add(outbuf, r, bbuf, r)
                return 0

            lax.fori_loop(0, SR // 2, init_body, 0)

            def batch_body(bi, _):
                s = pl.multiple_of(base + bi * 128, 8)
                pltpu.sync_copy(mpo_hbm.at[pl.ds(s, 128)], pob)
                pltpu.sync_copy(mps_hbm.at[pl.ds(s, 128)], psb.at[pl.ds(0, 128)])
                pltpu.sync_copy(y1_hbm.at[pob], ybuf)
                _clamp_offsets(psb, offb, ar, SR)

                def pair_body(g, _):
                    for u in range(2):
                        k = 2 * g + u
                        _row_add(outbuf, _sread(offb, k), ybuf, k)
                    return 0

                lax.fori_loop(0, 64, pair_body, 0)
                return 0

            lax.fori_loop(0, nb, batch_body, 0)

            @pl.when(c != last_c)
            def _():
                pltpu.sync_copy(outbuf.at[pl.ds(0, SR)], out_hbm.at[pl.ds(ar, SR)])

            @pl.when(c == last_c)
            def _():
                pltpu.sync_copy(outbuf.at[pl.ds(0, lastv)],
                                out_hbm.at[pl.ds(last_c * SR, lastv)])

            return 0

        lax.fori_loop(0, nch, chunk_body, 0)

    return outk


def kernel(x_values, W, b, x_indices, a_indices, mp_src, mp_out):
    nnz = x_values.shape[0]
    i0 = x_indices[0].astype(_i32)
    i1 = x_indices[1].astype(_i32)
    a_src = a_indices[0].astype(_i32)
    a_dst = a_indices[1].astype(_i32)
    mps = mp_src.astype(_i32)
    mpo = mp_out.astype(_i32)

    SRK4 = 256
    nch = (nnz + SRK4 - 1) // SRK4
    pps = jnp.searchsorted(mps, jnp.arange(0, (nch + 1) * SRK4, SRK4,
                                           dtype=_i32)).astype(_i32)
    pps = jnp.pad(pps, (0, 688 - pps.shape[0]))
    xbs = jnp.searchsorted(i0, jnp.arange(0, 11521, 120, dtype=_i32)).astype(_i32)
    xbs = jnp.pad(xbs, (0, 112 - xbs.shape[0]), constant_values=nnz)
    aps = jnp.searchsorted(a_src, jnp.arange(0, 10241, 320, dtype=_i32)).astype(_i32)
    aps = jnp.pad(aps, (0, 48 - aps.shape[0]), constant_values=a_src.shape[0])

    i0t = jnp.pad(i0, (0, PAD), constant_values=-1)
    i1t = jnp.pad(i1, (0, PAD), constant_values=-2)
    i0g = jnp.pad(i0, (0, PAD), constant_values=0)
    i1g = jnp.pad(i1, (0, PAD), constant_values=0)
    mpog = jnp.pad(mpo, (0, PAD), constant_values=0)
    mpst = jnp.pad(mps, (0, PAD), constant_values=-1)
    adg = jnp.pad(a_dst, (0, PAD), constant_values=0)
    ast = jnp.pad(a_src, (0, PAD), constant_values=-1)

    w1, w2, w3, w4, w5, w6 = (W[D * k:D * (k + 1)] for k in range(6))
    b2d = b.reshape(1, D)

    y1 = _tc_matmul(x_values, w1)

    pool1, p0parts, diag = _make_pools_kernel(nnz)(x_values, i0t, i1t, xbs)
    pool0 = _tc_add(p0parts[0], p0parts[1])

    x5 = _make_x5_kernel()(pool0, adg, ast, aps)

    g0, g1 = _tc_node_matmul(diag, pool1, pool0, x5, w2, w3, w4, w5, w6, b2d)

    out = _make_out_kernel(nnz)(g0, g1, y1, i0g, i1g, mpog, mpst, pps)
    return out


# speedup vs baseline: 1.1823x; 1.1823x over previous
import functools

import jax
import jax.numpy as jnp
from jax import lax
from jax.experimental import pallas as pl
from jax.experimental.pallas import tpu as pltpu
from jax.experimental.pallas import tpu_sc as plsc

N = 10000
D = 128
L = 16
NT = 32
PAD = 640

_f32 = jnp.float32
_i32 = jnp.int32


def _wid():
    return lax.axis_index("s") * 2 + lax.axis_index("c")


def _sread(ref, idx):
    return ref[pl.ds(idx, L)][0]


def _clamp_offsets(idxbuf, offbuf, base_val, limit):
    for j in range(128 // L):
        off = idxbuf[pl.ds(L * j, L)] - base_val
        ok = (off >= 0) & (off < limit)
        offbuf[pl.ds(L * j, L)] = jnp.where(ok, off, limit)


def _row_add(dst, dst_row, src, src_row):
    for j in range(D // L):
        v = src[src_row, pl.ds(L * j, L)]
        plsc.addupdate(dst.at[dst_row, pl.ds(L * j, L)], v)


def _zero_rows(buf, nrows):
    z = jnp.zeros((L,), _f32)

    def body(r, _):
        for j in range(D // L):
            buf[r, pl.ds(L * j, L)] = z
        return 0

    lax.fori_loop(0, nrows, body, 0)


def _mm_body(x_ref, w_ref, o_ref):
    o_ref[...] = jnp.dot(x_ref[...], w_ref[...], preferred_element_type=_f32)


def _tc_matmul(x, w):
    nnz = x.shape[0]
    br = 2048
    g = (nnz + br - 1) // br
    return pl.pallas_call(
        _mm_body,
        grid=(g,),
        in_specs=[
            pl.BlockSpec((br, D), lambda i: (i, 0)),
            pl.BlockSpec((D, D), lambda i: (0, 0)),
        ],
        out_specs=pl.BlockSpec((br, D), lambda i: (i, 0)),
        out_shape=jax.ShapeDtypeStruct((nnz, D), _f32),
    )(x, w)


def _node_mm_body(d_ref, p1_ref, p0_ref, x5_ref, w2, w3, w4, w5, w6, b_ref,
                  g0_ref, g1_ref):
    dd = d_ref[...]
    g0_ref[...] = (jnp.dot(dd, w2[...], preferred_element_type=_f32)
                   + jnp.dot(p0_ref[...], w5[...], preferred_element_type=_f32)
                   + jnp.dot(x5_ref[...], w6[...], preferred_element_type=_f32)
                   + b_ref[...])
    g1_ref[...] = (jnp.dot(dd, w3[...], preferred_element_type=_f32)
                   + jnp.dot(p1_ref[...], w4[...], preferred_element_type=_f32))


def _tc_node_matmul(diag, pool1, pool0, x5, w2, w3, w4, w5, w6, b2d):
    br = 1000
    g = N // br
    full = pl.BlockSpec((D, D), lambda i: (0, 0))
    blk = pl.BlockSpec((br, D), lambda i: (i, 0))
    return pl.pallas_call(
        _node_mm_body,
        grid=(g,),
        in_specs=[blk, blk, blk, blk, full, full, full, full, full,
                  pl.BlockSpec((1, D), lambda i: (0, 0))],
        out_specs=[blk, blk],
        out_shape=[jax.ShapeDtypeStruct((N, D), _f32),
                   jax.ShapeDtypeStruct((N, D), _f32)],
    )(diag, pool1, pool0, x5, w2, w3, w4, w5, w6, b2d)


def _make_pools_kernel(nnz):
    mesh = plsc.VectorSubcoreMesh(core_axis_name="c", subcore_axis_name="s")
    SN = 120
    NS = 10016
    ZR = 624

    @functools.partial(
        pl.kernel,
        out_type=[jax.ShapeDtypeStruct((N, D), _f32),
                  jax.ShapeDtypeStruct((2, N, D), _f32),
                  jax.ShapeDtypeStruct((N, D), _f32)],
        mesh=mesh,
        scratch_types=[
            pltpu.VMEM((SN + 1, D), _f32),
            pltpu.VMEM((SN + 1, D), _f32),
            pltpu.VMEM((128, D), _f32),
            pltpu.VMEM((128,), _i32),
            pltpu.VMEM((128,), _i32),
            pltpu.VMEM((128,), _i32),
            pltpu.VMEM((128,), _i32),
            pltpu.VMEM((128,), _i32),
            pltpu.VMEM((128,), _i32),
            pltpu.VMEM((112,), _i32),
            pltpu.VMEM_SHARED((NS, D), _f32),
        ],
    )
    def pools(x_hbm, i0t_hbm, i1t_hbm, xbs_hbm,
              p1_hbm, p0p_hbm, dg_hbm,
              p1buf, dbuf, xbuf, i0b, i1b, i1c, idb, offb, dfb, xbsb, spool):
        cid = lax.axis_index("c")
        sid = lax.axis_index("s")
        wid = sid * 2 + cid

        pltpu.sync_copy(xbs_hbm, xbsb)

        _zero_rows(xbuf, 128)
        z0 = pl.multiple_of(sid * ZR, 8)
        for h in range(4):
            pltpu.sync_copy(xbuf.at[pl.ds(0, 128)],
                            spool.at[pl.ds(z0 + 128 * h, 128)])

        @pl.when(sid < 15)
        def _():
            pltpu.sync_copy(xbuf.at[pl.ds(0, 112)],
                            spool.at[pl.ds(z0 + 512, 112)])

        @pl.when(sid == 15)
        def _():
            pltpu.sync_copy(xbuf.at[pl.ds(0, 128)],
                            spool.at[pl.ds(z0 + 512, 128)])

        plsc.subcore_barrier()

        def chunk_body(ci, _):
            n0 = pl.multiple_of(wid * (3 * SN) + ci * SN, 8)
            r0 = _sread(xbsb, 3 * wid + ci)
            r1 = _sread(xbsb, 3 * wid + ci + 1)
            base = r0 & ~7
            nb = (r1 - base + 127) // 128

            _zero_rows(p1buf, SN + 1)
            _zero_rows(dbuf, SN + 1)

            def batch_body(bi, _):
                s = pl.multiple_of(base + bi * 128, 8)
                pltpu.sync_copy(i0t_hbm.at[pl.ds(s, 128)], i0b)
                pltpu.sync_copy(i1t_hbm.at[pl.ds(s, 128)], i1b)
                for j in range(128 // L):
                    v = s + L * j + lax.iota(_i32, L)
                    idb[pl.ds(L * j, L)] = jnp.minimum(v, nnz - 1)
                pltpu.sync_copy(x_hbm.at[idb], xbuf)
                for j in range(128 // L):
                    pos = s + L * j + lax.iota(_i32, L)
                    okp = (pos >= r0) & (pos < r1)
                    i1v = i1b[pl.ds(L * j, L)]
                    i1c[pl.ds(L * j, L)] = jnp.where(okp, i1v, N)
                pltpu.sync_copy(xbuf, spool.at[i1c], add=True)

                _clamp_offsets(i0b, offb, n0, SN)
                for j in range(128 // L):
                    offv = offb[pl.ds(L * j, L)]
                    eq = i0b[pl.ds(L * j, L)] == i1b[pl.ds(L * j, L)]
                    dfb[pl.ds(L * j, L)] = jnp.where(eq, offv, SN)

                def row_body(g, _):
                    gb = g * L
                    offv = offb[pl.ds(gb, L)]
                    dfv = dfb[pl.ds(gb, L)]
                    for u in range(L):
                        _row_add(p1buf, offv[u], xbuf, gb + u)
                        dof = dfv[u]

                        @pl.when(dof < SN)
                        def _():
                            _row_add(dbuf, dof, xbuf, gb + u)

                    return 0

                lax.fori_loop(0, 128 // L, row_body, 0)
                return 0

            lax.fori_loop(0, nb, batch_body, 0)

            full = n0 + SN <= N
            part = n0 == (N // SN) * SN

            @pl.when(full)
            def _():
                pltpu.sync_copy(p1buf.at[pl.ds(0, SN)], p1_hbm.at[pl.ds(n0, SN)])
                pltpu.sync_copy(dbuf.at[pl.ds(0, SN)], dg_hbm.at[pl.ds(n0, SN)])

            @pl.when(part)
            def _():
                rem = N - (N // SN) * SN
                pltpu.sync_copy(p1buf.at[pl.ds(0, rem)], p1_hbm.at[pl.ds(n0, rem)])
                pltpu.sync_copy(dbuf.at[pl.ds(0, rem)], dg_hbm.at[pl.ds(n0, rem)])

            return 0

        lax.fori_loop(0, 3, chunk_body, 0)

        plsc.subcore_barrier()
        w0 = pl.multiple_of(sid * ZR, 8)

        @pl.when(sid < 15)
        def _():
            pltpu.sync_copy(spool.at[pl.ds(w0, ZR)],
                            p0p_hbm.at[cid, pl.ds(w0, ZR)])

        @pl.when(sid == 15)
        def _():
            pltpu.sync_copy(spool.at[pl.ds(w0, 640)],
                            p0p_hbm.at[cid, pl.ds(w0, 640)])

    return pools


def _add_body(a_ref, b_ref, o_ref):
    o_ref[...] = a_ref[...] + b_ref[...]


def _tc_add(a, b):
    br = 1000
    blk = pl.BlockSpec((br, D), lambda i: (i, 0))
    return pl.pallas_call(
        _add_body,
        grid=(N // br,),
        in_specs=[blk, blk],
        out_specs=blk,
        out_shape=jax.ShapeDtypeStruct((N, D), _f32),
    )(a, b)


def _make_x5_kernel():
    mesh = plsc.VectorSubcoreMesh(core_axis_name="c", subcore_axis_name="s")
    SN = 320

    @functools.partial(
        pl.kernel,
        out_type=jax.ShapeDtypeStruct((N, D), _f32),
        mesh=mesh,
        scratch_types=[
            pltpu.VMEM((SN + 1, D), _f32),
            pltpu.VMEM((128, D), _f32),
            pltpu.VMEM((128,), _i32),
            pltpu.VMEM((128,), _i32),
            pltpu.VMEM((128,), _i32),
            pltpu.VMEM((48,), _i32),
        ],
    )
    def x5k(p0_hbm, adst_hbm, asrc_hbm, ap_hbm, x5_hbm,
            x5buf, gbuf, adb, asb, offb, apb):
        wid = _wid()
        n0 = pl.multiple_of(wid * SN, 8)
        pltpu.sync_copy(ap_hbm, apb)
        e0 = _sread(apb, wid)
        e1 = _sread(apb, wid + 1)
        base = e0 & ~7
        nb = (e1 - base + 127) // 128

        _zero_rows(x5buf, SN + 1)

        def batch_body(bi, _):
            s = pl.multiple_of(base + bi * 128, 8)
            pltpu.sync_copy(adst_hbm.at[pl.ds(s, 128)], adb)
            pltpu.sync_copy(asrc_hbm.at[pl.ds(s, 128)], asb)
            pltpu.sync_copy(p0_hbm.at[adb], gbuf)
            _clamp_offsets(asb, offb, n0, SN)

            def edge_body(g, _):
                gb = g * L
                offv = offb[pl.ds(gb, L)]
                for u in range(L):
                    _row_add(x5buf, offv[u], gbuf, gb + u)
                return 0

            lax.fori_loop(0, 128 // L, edge_body, 0)
            return 0

        lax.fori_loop(0, nb, batch_body, 0)

        full = n0 + SN <= N
        part = n0 == (N // SN) * SN

        @pl.when(full)
        def _():
            pltpu.sync_copy(x5buf.at[pl.ds(0, SN)], x5_hbm.at[pl.ds(n0, SN)])

        @pl.when(part)
        def _():
            rem = N - (N // SN) * SN
            pltpu.sync_copy(x5buf.at[pl.ds(0, rem)], x5_hbm.at[pl.ds(n0, rem)])

    return x5k


def _make_out_kernel(nnz):
    mesh = plsc.VectorSubcoreMesh(core_axis_name="c", subcore_axis_name="s")
    SR = 256
    nch_total = (nnz + SR - 1) // SR
    last_c = nch_total - 1
    lastv = nnz - last_c * SR
    base_nch = nch_total // NT
    extra = nch_total - base_nch * NT

    @functools.partial(
        pl.kernel,
        out_type=jax.ShapeDtypeStruct((nnz, D), _f32),
        mesh=mesh,
        scratch_types=[
            pltpu.VMEM((SR + 1, D), _f32),
            pltpu.VMEM((SR, D), _f32),
            pltpu.VMEM((128, D), _f32),
            pltpu.VMEM((128,), _i32),
            pltpu.VMEM((128,), _i32),
            pltpu.VMEM((128,), _i32),
            pltpu.VMEM((128,), _i32),
            pltpu.VMEM((128,), _i32),
            pltpu.VMEM((128,), _i32),
            pltpu.VMEM((128,), _i32),
            pltpu.VMEM((688,), _i32),
            pltpu.SemaphoreType.DMA,
            pltpu.SemaphoreType.DMA,
            pltpu.SemaphoreType.DMA,
            pltpu.SemaphoreType.DMA,
        ],
    )
    def outk(g0_hbm, g1_hbm, y1_hbm, i0_hbm, i1_hbm, mpo_hbm, mps_hbm, pp_hbm,
             out_hbm, outbuf, bbuf, ybuf, ib0, ib1, ib2, ib3, pob, psb, offb,
             ppb, sem0, sem1, sem2, sem3):
        wid = _wid()
        nch = base_nch + jnp.where(wid < extra, 1, 0)
        pltpu.sync_copy(pp_hbm, ppb)

        def chunk_body(ci, _):
            c = wid + NT * ci
            ar = pl.multiple_of(c * SR, 8)
            p0 = _sread(ppb, c)
            p1 = _sread(ppb, c + 1)
            base = p0 & ~7
            nb = (p1 - base + 127) // 128

            pltpu.sync_copy(i0_hbm.at[pl.ds(ar, 128)], ib0)
            pltpu.sync_copy(i0_hbm.at[pl.ds(ar + 128, 128)], ib1)
            pltpu.sync_copy(i1_hbm.at[pl.ds(ar, 128)], ib2)
            pltpu.sync_copy(i1_hbm.at[pl.ds(ar + 128, 128)], ib3)
            c0 = pltpu.async_copy(g0_hbm.at[ib0], outbuf.at[pl.ds(0, 128)], sem0)
            c1 = pltpu.async_copy(g0_hbm.at[ib1], outbuf.at[pl.ds(128, 128)], sem1)
            c2 = pltpu.async_copy(g1_hbm.at[ib2], bbuf.at[pl.ds(0, 128)], sem2)
            c3 = pltpu.async_copy(g1_hbm.at[ib3], bbuf.at[pl.ds(128, 128)], sem3)
            c0.wait()
            c1.wait()
            c2.wait()
            c3.wait()

            def init_body(g, _):
                for u in range(2):
                    r = 2 * g + u
                    _row_add(outbuf, r, bbuf, r)
                return 0

            lax.fori_loop(0, SR // 2, init_body, 0)

            def batch_body(bi, _):
                s = pl.multiple_of(base + bi * 128, 8)
                pltpu.sync_copy(mpo_hbm.at[pl.ds(s, 128)], pob)
                pltpu.sync_copy(mps_hbm.at[pl.ds(s, 128)], psb)
                pltpu.sync_copy(y1_hbm.at[pob], ybuf)
                _clamp_offsets(psb, offb, ar, SR)

                def pair_body(g, _):
                    gb = g * L
                    offv = offb[pl.ds(gb, L)]
                    for u in range(L):
                        _row_add(outbuf, offv[u], ybuf, gb + u)
                    return 0

                lax.fori_loop(0, 128 // L, pair_body, 0)
                return 0

            lax.fori_loop(0, nb, batch_body, 0)

            @pl.when(c != last_c)
            def _():
                pltpu.sync_copy(outbuf.at[pl.ds(0, SR)], out_hbm.at[pl.ds(ar, SR)])

            @pl.when(c == last_c)
            def _():
                pltpu.sync_copy(outbuf.at[pl.ds(0, lastv)],
                                out_hbm.at[pl.ds(last_c * SR, lastv)])

            return 0

        lax.fori_loop(0, nch, chunk_body, 0)

    return outk


def kernel(x_values, W, b, x_indices, a_indices, mp_src, mp_out):
    nnz = x_values.shape[0]
    i0 = x_indices[0].astype(_i32)
    i1 = x_indices[1].astype(_i32)
    a_src = a_indices[0].astype(_i32)
    a_dst = a_indices[1].astype(_i32)
    mps = mp_src.astype(_i32)
    mpo = mp_out.astype(_i32)

    SRK4 = 256
    nch = (nnz + SRK4 - 1) // SRK4
    pps = jnp.searchsorted(mps, jnp.arange(0, (nch + 1) * SRK4, SRK4,
                                           dtype=_i32)).astype(_i32)
    pps = jnp.pad(pps, (0, 688 - pps.shape[0]))
    xbs = jnp.searchsorted(i0, jnp.arange(0, 11521, 120, dtype=_i32)).astype(_i32)
    xbs = jnp.pad(xbs, (0, 112 - xbs.shape[0]), constant_values=nnz)
    aps = jnp.searchsorted(a_src, jnp.arange(0, 10241, 320, dtype=_i32)).astype(_i32)
    aps = jnp.pad(aps, (0, 48 - aps.shape[0]), constant_values=a_src.shape[0])

    i0t = jnp.pad(i0, (0, PAD), constant_values=-1)
    i1t = jnp.pad(i1, (0, PAD), constant_values=-2)
    i0g = jnp.pad(i0, (0, PAD), constant_values=0)
    i1g = jnp.pad(i1, (0, PAD), constant_values=0)
    mpog = jnp.pad(mpo, (0, PAD), constant_values=0)
    mpst = jnp.pad(mps, (0, PAD), constant_values=-1)
    adg = jnp.pad(a_dst, (0, PAD), constant_values=0)
    ast = jnp.pad(a_src, (0, PAD), constant_values=-1)

    w1, w2, w3, w4, w5, w6 = (W[D * k:D * (k + 1)] for k in range(6))
    b2d = b.reshape(1, D)

    y1 = _tc_matmul(x_values, w1)

    pool1, p0parts, diag = _make_pools_kernel(nnz)(x_values, i0t, i1t, xbs)
    pool0 = _tc_add(p0parts[0], p0parts[1])

    x5 = _make_x5_kernel()(pool0, adg, ast, aps)

    g0, g1 = _tc_node_matmul(diag, pool1, pool0, x5, w2, w3, w4, w5, w6, b2d)

    out = _make_out_kernel(nnz)(g0, g1, y1, i0g, i1g, mpog, mpst, pps)
    return out


# speedup vs baseline: 1.4024x; 1.1862x over previous
import functools

import jax
import jax.numpy as jnp
from jax import lax
from jax.experimental import pallas as pl
from jax.experimental.pallas import tpu as pltpu
from jax.experimental.pallas import tpu_sc as plsc

N = 10000
D = 128
L = 16
NT = 32
PAD = 640

_f32 = jnp.float32
_i32 = jnp.int32


def _wid():
    return lax.axis_index("s") * 2 + lax.axis_index("c")


def _sread(ref, idx):
    return ref[pl.ds(idx, L)][0]


def _clamp_offsets(idxbuf, offbuf, base_val, limit):
    for j in range(128 // L):
        off = idxbuf[pl.ds(L * j, L)] - base_val
        ok = (off >= 0) & (off < limit)
        offbuf[pl.ds(L * j, L)] = jnp.where(ok, off, limit)


def _row_add(dst, dst_row, src, src_row):
    for j in range(D // L):
        v = src[src_row, pl.ds(L * j, L)]
        plsc.addupdate(dst.at[dst_row, pl.ds(L * j, L)], v)


def _zero_rows(buf, nrows):
    z = jnp.zeros((L,), _f32)

    def body(r, _):
        for j in range(D // L):
            buf[r, pl.ds(L * j, L)] = z
        return 0

    lax.fori_loop(0, nrows, body, 0)


def _mm_body(x_ref, w_ref, o_ref):
    o_ref[...] = jnp.dot(x_ref[...], w_ref[...], preferred_element_type=_f32)


def _tc_matmul(x, w):
    nnz = x.shape[0]
    br = 2048
    g = (nnz + br - 1) // br
    return pl.pallas_call(
        _mm_body,
        grid=(g,),
        in_specs=[
            pl.BlockSpec((br, D), lambda i: (i, 0)),
            pl.BlockSpec((D, D), lambda i: (0, 0)),
        ],
        out_specs=pl.BlockSpec((br, D), lambda i: (i, 0)),
        out_shape=jax.ShapeDtypeStruct((nnz, D), _f32),
    )(x, w)


def _node_mm_body(d_ref, p1_ref, p0_ref, x5_ref, w2, w3, w4, w5, w6, b_ref,
                  g0_ref, g1_ref):
    dd = d_ref[...]
    g0_ref[...] = (jnp.dot(dd, w2[...], preferred_element_type=_f32)
                   + jnp.dot(p0_ref[...], w5[...], preferred_element_type=_f32)
                   + jnp.dot(x5_ref[...], w6[...], preferred_element_type=_f32)
                   + b_ref[...])
    g1_ref[...] = (jnp.dot(dd, w3[...], preferred_element_type=_f32)
                   + jnp.dot(p1_ref[...], w4[...], preferred_element_type=_f32))


def _tc_node_matmul(diag, pool1, pool0, x5, w2, w3, w4, w5, w6, b2d):
    br = 1000
    g = N // br
    full = pl.BlockSpec((D, D), lambda i: (0, 0))
    blk = pl.BlockSpec((br, D), lambda i: (i, 0))
    return pl.pallas_call(
        _node_mm_body,
        grid=(g,),
        in_specs=[blk, blk, blk, blk, full, full, full, full, full,
                  pl.BlockSpec((1, D), lambda i: (0, 0))],
        out_specs=[blk, blk],
        out_shape=[jax.ShapeDtypeStruct((N, D), _f32),
                   jax.ShapeDtypeStruct((N, D), _f32)],
    )(diag, pool1, pool0, x5, w2, w3, w4, w5, w6, b2d)


def _make_pools_kernel(nnz):
    mesh = plsc.VectorSubcoreMesh(core_axis_name="c", subcore_axis_name="s")
    SN = 120
    NS = 10016
    ZR = 624

    @functools.partial(
        pl.kernel,
        out_type=[jax.ShapeDtypeStruct((N, D), _f32),
                  jax.ShapeDtypeStruct((2, N, D), _f32),
                  jax.ShapeDtypeStruct((N, D), _f32)],
        mesh=mesh,
        scratch_types=[
            pltpu.VMEM((SN + 1, D), _f32),
            pltpu.VMEM((SN + 1, D), _f32),
            pltpu.VMEM((128, D), _f32),
            pltpu.VMEM((128,), _i32),
            pltpu.VMEM((128,), _i32),
            pltpu.VMEM((128,), _i32),
            pltpu.VMEM((128,), _i32),
            pltpu.VMEM((128,), _i32),
            pltpu.VMEM((128,), _i32),
            pltpu.VMEM((112,), _i32),
            pltpu.VMEM_SHARED((NS, D), _f32),
        ],
    )
    def pools(x_hbm, i0t_hbm, i1t_hbm, xbs_hbm,
              p1_hbm, p0p_hbm, dg_hbm,
              p1buf, dbuf, xbuf, i0b, i1b, i1c, idb, offb, dfb, xbsb, spool):
        cid = lax.axis_index("c")
        sid = lax.axis_index("s")
        wid = sid * 2 + cid

        pltpu.sync_copy(xbs_hbm, xbsb)

        _zero_rows(xbuf, 128)
        z0 = pl.multiple_of(sid * ZR, 8)
        for h in range(4):
            pltpu.sync_copy(xbuf.at[pl.ds(0, 128)],
                            spool.at[pl.ds(z0 + 128 * h, 128)])

        @pl.when(sid < 15)
        def _():
            pltpu.sync_copy(xbuf.at[pl.ds(0, 112)],
                            spool.at[pl.ds(z0 + 512, 112)])

        @pl.when(sid == 15)
        def _():
            pltpu.sync_copy(xbuf.at[pl.ds(0, 128)],
                            spool.at[pl.ds(z0 + 512, 128)])

        plsc.subcore_barrier()

        def chunk_body(ci, _):
            n0 = pl.multiple_of(wid * (3 * SN) + ci * SN, 8)
            r0 = _sread(xbsb, 3 * wid + ci)
            r1 = _sread(xbsb, 3 * wid + ci + 1)
            base = r0 & ~7
            nb = (r1 - base + 127) // 128

            _zero_rows(p1buf, SN + 1)
            _zero_rows(dbuf, SN + 1)

            def batch_body(bi, _):
                s = pl.multiple_of(base + bi * 128, 8)
                pltpu.sync_copy(i0t_hbm.at[pl.ds(s, 128)], i0b)
                pltpu.sync_copy(i1t_hbm.at[pl.ds(s, 128)], i1b)
                for j in range(128 // L):
                    v = s + L * j + lax.iota(_i32, L)
                    idb[pl.ds(L * j, L)] = jnp.minimum(v, nnz - 1)
                pltpu.sync_copy(x_hbm.at[idb], xbuf)
                for j in range(128 // L):
                    pos = s + L * j + lax.iota(_i32, L)
                    okp = (pos >= r0) & (pos < r1)
                    i1v = i1b[pl.ds(L * j, L)]
                    i1c[pl.ds(L * j, L)] = jnp.where(okp, i1v, N)
                pltpu.sync_copy(xbuf, spool.at[i1c], add=True)

                _clamp_offsets(i0b, offb, n0, SN)
                for j in range(128 // L):
                    offv = offb[pl.ds(L * j, L)]
                    eq = i0b[pl.ds(L * j, L)] == i1b[pl.ds(L * j, L)]
                    dfb[pl.ds(L * j, L)] = jnp.where(eq, offv, SN)

                def row_body(g, _):
                    gb = g * L
                    offv = offb[pl.ds(gb, L)]
                    dfv = dfb[pl.ds(gb, L)]
                    for u in range(L):
                        _row_add(p1buf, offv[u], xbuf, gb + u)
                        dof = dfv[u]

                        @pl.when(dof < SN)
                        def _():
                            _row_add(dbuf, dof, xbuf, gb + u)

                    return 0

                lax.fori_loop(0, 128 // L, row_body, 0)
                return 0

            lax.fori_loop(0, nb, batch_body, 0)

            full = n0 + SN <= N
            part = n0 == (N // SN) * SN

            @pl.when(full)
            def _():
                pltpu.sync_copy(p1buf.at[pl.ds(0, SN)], p1_hbm.at[pl.ds(n0, SN)])
                pltpu.sync_copy(dbuf.at[pl.ds(0, SN)], dg_hbm.at[pl.ds(n0, SN)])

            @pl.when(part)
            def _():
                rem = N - (N // SN) * SN
                pltpu.sync_copy(p1buf.at[pl.ds(0, rem)], p1_hbm.at[pl.ds(n0, rem)])
                pltpu.sync_copy(dbuf.at[pl.ds(0, rem)], dg_hbm.at[pl.ds(n0, rem)])

            return 0

        lax.fori_loop(0, 3, chunk_body, 0)

        plsc.subcore_barrier()
        w0 = pl.multiple_of(sid * ZR, 8)

        @pl.when(sid < 15)
        def _():
            pltpu.sync_copy(spool.at[pl.ds(w0, ZR)],
                            p0p_hbm.at[cid, pl.ds(w0, ZR)])

        @pl.when(sid == 15)
        def _():
            pltpu.sync_copy(spool.at[pl.ds(w0, 640)],
                            p0p_hbm.at[cid, pl.ds(w0, 640)])

    return pools


def _add_body(a_ref, b_ref, o_ref):
    o_ref[...] = a_ref[...] + b_ref[...]


def _tc_add(a, b):
    br = 1000
    blk = pl.BlockSpec((br, D), lambda i: (i, 0))
    return pl.pallas_call(
        _add_body,
        grid=(N // br,),
        in_specs=[blk, blk],
        out_specs=blk,
        out_shape=jax.ShapeDtypeStruct((N, D), _f32),
    )(a, b)


def _make_x5_kernel():
    mesh = plsc.VectorSubcoreMesh(core_axis_name="c", subcore_axis_name="s")
    SN = 320

    @functools.partial(
        pl.kernel,
        out_type=jax.ShapeDtypeStruct((N, D), _f32),
        mesh=mesh,
        scratch_types=[
            pltpu.VMEM((SN + 1, D), _f32),
            pltpu.VMEM((128, D), _f32),
            pltpu.VMEM((128, D), _f32),
            pltpu.VMEM((128,), _i32),
            pltpu.VMEM((128,), _i32),
            pltpu.VMEM((128,), _i32),
            pltpu.VMEM((128,), _i32),
            pltpu.VMEM((128,), _i32),
            pltpu.VMEM((48,), _i32),
            pltpu.SemaphoreType.DMA,
            pltpu.SemaphoreType.DMA,
        ],
    )
    def x5k(p0_hbm, adst_hbm, asrc_hbm, ap_hbm, x5_hbm,
            x5buf, gbufa, gbufb, adba, adbb, asba, asbb, offb, apb,
            sema, semb):
        wid = _wid()
        n0 = pl.multiple_of(wid * SN, 8)
        pltpu.sync_copy(ap_hbm, apb)
        e0 = _sread(apb, wid)
        e1 = _sread(apb, wid + 1)
        base = e0 & ~7
        nb = (e1 - base + 127) // 128

        def issue(bi, adb, asb, gbuf, sem):
            st = pl.multiple_of(base + bi * 128, 8)
            pltpu.sync_copy(adst_hbm.at[pl.ds(st, 128)], adb)
            pltpu.sync_copy(asrc_hbm.at[pl.ds(st, 128)], asb)
            pltpu.async_copy(p0_hbm.at[adb], gbuf, sem)

        def process(asb, gbuf):
            _clamp_offsets(asb, offb, n0, SN)

            def edge_body(g, _):
                gb = g * L
                offv = offb[pl.ds(gb, L)]
                for u in range(L):
                    _row_add(x5buf, offv[u], gbuf, gb + u)
                return 0

            lax.fori_loop(0, 128 // L, edge_body, 0)

        _zero_rows(x5buf, SN + 1)

        @pl.when(nb > 0)
        def _():
            issue(0, adba, asba, gbufa, sema)

        def batch_body(bi, _):
            even = (bi % 2) == 0

            @pl.when(even)
            def _():
                pltpu.make_async_copy(p0_hbm.at[adba], gbufa, sema).wait()

                @pl.when(bi + 1 < nb)
                def _():
                    issue(bi + 1, adbb, asbb, gbufb, semb)

                process(asba, gbufa)

            @pl.when(~even)
            def _():
                pltpu.make_async_copy(p0_hbm.at[adbb], gbufb, semb).wait()

                @pl.when(bi + 1 < nb)
                def _():
                    issue(bi + 1, adba, asba, gbufa, sema)

                process(asbb, gbufb)

            return 0

        lax.fori_loop(0, nb, batch_body, 0)

        full = n0 + SN <= N
        part = n0 == (N // SN) * SN

        @pl.when(full)
        def _():
            pltpu.sync_copy(x5buf.at[pl.ds(0, SN)], x5_hbm.at[pl.ds(n0, SN)])

        @pl.when(part)
        def _():
            rem = N - (N // SN) * SN
            pltpu.sync_copy(x5buf.at[pl.ds(0, rem)], x5_hbm.at[pl.ds(n0, rem)])

    return x5k


def _make_out_kernel(nnz):
    mesh = plsc.VectorSubcoreMesh(core_axis_name="c", subcore_axis_name="s")
    SR = 256
    nch_total = (nnz + SR - 1) // SR
    last_c = nch_total - 1
    lastv = nnz - last_c * SR
    base_nch = nch_total // NT
    extra = nch_total - base_nch * NT

    @functools.partial(
        pl.kernel,
        out_type=jax.ShapeDtypeStruct((nnz, D), _f32),
        mesh=mesh,
        scratch_types=[
            pltpu.VMEM((SR + 1, D), _f32),
            pltpu.VMEM((SR, D), _f32),
            pltpu.VMEM((128, D), _f32),
            pltpu.VMEM((128, D), _f32),
            pltpu.VMEM((128,), _i32),
            pltpu.VMEM((128,), _i32),
            pltpu.VMEM((128,), _i32),
            pltpu.VMEM((128,), _i32),
            pltpu.VMEM((128,), _i32),
            pltpu.VMEM((128,), _i32),
            pltpu.VMEM((128,), _i32),
            pltpu.VMEM((128,), _i32),
            pltpu.VMEM((128,), _i32),
            pltpu.VMEM((688,), _i32),
            pltpu.SemaphoreType.DMA,
            pltpu.SemaphoreType.DMA,
            pltpu.SemaphoreType.DMA,
            pltpu.SemaphoreType.DMA,
            pltpu.SemaphoreType.DMA,
            pltpu.SemaphoreType.DMA,
        ],
    )
    def outk(g0_hbm, g1_hbm, y1_hbm, i0_hbm, i1_hbm, mpo_hbm, mps_hbm, pp_hbm,
             out_hbm, outbuf, bbuf, ybufa, ybufb, ib0, ib1, ib2, ib3,
             poba, pobb, psba, psbb, offb, ppb, sem0, sem1, sem2, sem3,
             sema, semb):
        wid = _wid()
        nch = base_nch + jnp.where(wid < extra, 1, 0)
        pltpu.sync_copy(pp_hbm, ppb)

        def issue(base, bi, pob, psb, ybuf, sem):
            st = pl.multiple_of(base + bi * 128, 8)
            pltpu.sync_copy(mpo_hbm.at[pl.ds(st, 128)], pob)
            pltpu.sync_copy(mps_hbm.at[pl.ds(st, 128)], psb)
            pltpu.async_copy(y1_hbm.at[pob], ybuf, sem)

        def process(ar, psb, ybuf):
            _clamp_offsets(psb, offb, ar, SR)

            def pair_body(g, _):
                gb = g * L
                offv = offb[pl.ds(gb, L)]
                for u in range(L):
                    _row_add(outbuf, offv[u], ybuf, gb + u)
                return 0

            lax.fori_loop(0, 128 // L, pair_body, 0)

        def chunk_body(ci, _):
            c = wid + NT * ci
            ar = pl.multiple_of(c * SR, 8)
            p0 = _sread(ppb, c)
            p1 = _sread(ppb, c + 1)
            base = p0 & ~7
            nb = (p1 - base + 127) // 128

            pltpu.sync_copy(i0_hbm.at[pl.ds(ar, 128)], ib0)
            pltpu.sync_copy(i0_hbm.at[pl.ds(ar + 128, 128)], ib1)
            pltpu.sync_copy(i1_hbm.at[pl.ds(ar, 128)], ib2)
            pltpu.sync_copy(i1_hbm.at[pl.ds(ar + 128, 128)], ib3)
            c0 = pltpu.async_copy(g0_hbm.at[ib0], outbuf.at[pl.ds(0, 128)], sem0)
            c1 = pltpu.async_copy(g0_hbm.at[ib1], outbuf.at[pl.ds(128, 128)], sem1)
            c2 = pltpu.async_copy(g1_hbm.at[ib2], bbuf.at[pl.ds(0, 128)], sem2)
            c3 = pltpu.async_copy(g1_hbm.at[ib3], bbuf.at[pl.ds(128, 128)], sem3)

            @pl.when(nb > 0)
            def _():
                issue(base, 0, poba, psba, ybufa, sema)

            c0.wait()
            c1.wait()
            c2.wait()
            c3.wait()

            def init_body(g, _):
                for u in range(2):
                    r = 2 * g + u
                    _row_add(outbuf, r, bbuf, r)
                return 0

            lax.fori_loop(0, SR // 2, init_body, 0)

            def batch_body(bi, _):
                even = (bi % 2) == 0

                @pl.when(even)
                def _():
                    pltpu.make_async_copy(y1_hbm.at[poba], ybufa, sema).wait()

                    @pl.when(bi + 1 < nb)
                    def _():
                        issue(base, bi + 1, pobb, psbb, ybufb, semb)

                    process(ar, psba, ybufa)

                @pl.when(~even)
                def _():
                    pltpu.make_async_copy(y1_hbm.at[pobb], ybufb, semb).wait()

                    @pl.when(bi + 1 < nb)
                    def _():
                        issue(base, bi + 1, poba, psba, ybufa, sema)

                    process(ar, psbb, ybufb)

                return 0

            lax.fori_loop(0, nb, batch_body, 0)

            @pl.when(c != last_c)
            def _():
                pltpu.sync_copy(outbuf.at[pl.ds(0, SR)], out_hbm.at[pl.ds(ar, SR)])

            @pl.when(c == last_c)
            def _():
                pltpu.sync_copy(outbuf.at[pl.ds(0, lastv)],
                                out_hbm.at[pl.ds(last_c * SR, lastv)])

            return 0

        lax.fori_loop(0, nch, chunk_body, 0)

    return outk


def kernel(x_values, W, b, x_indices, a_indices, mp_src, mp_out):
    nnz = x_values.shape[0]
    i0 = x_indices[0].astype(_i32)
    i1 = x_indices[1].astype(_i32)
    a_src = a_indices[0].astype(_i32)
    a_dst = a_indices[1].astype(_i32)
    mps = mp_src.astype(_i32)
    mpo = mp_out.astype(_i32)

    SRK4 = 256
    nch = (nnz + SRK4 - 1) // SRK4
    pps = jnp.searchsorted(mps, jnp.arange(0, (nch + 1) * SRK4, SRK4,
                                           dtype=_i32)).astype(_i32)
    pps = jnp.pad(pps, (0, 688 - pps.shape[0]))
    xbs = jnp.searchsorted(i0, jnp.arange(0, 11521, 120, dtype=_i32)).astype(_i32)
    xbs = jnp.pad(xbs, (0, 112 - xbs.shape[0]), constant_values=nnz)
    aps = jnp.searchsorted(a_src, jnp.arange(0, 10241, 320, dtype=_i32)).astype(_i32)
    aps = jnp.pad(aps, (0, 48 - aps.shape[0]), constant_values=a_src.shape[0])

    i0t = jnp.pad(i0, (0, PAD), constant_values=-1)
    i1t = jnp.pad(i1, (0, PAD), constant_values=-2)
    i0g = jnp.pad(i0, (0, PAD), constant_values=0)
    i1g = jnp.pad(i1, (0, PAD), constant_values=0)
    mpog = jnp.pad(mpo, (0, PAD), constant_values=0)
    mpst = jnp.pad(mps, (0, PAD), constant_values=-1)
    adg = jnp.pad(a_dst, (0, PAD), constant_values=0)
    ast = jnp.pad(a_src, (0, PAD), constant_values=-1)

    w1, w2, w3, w4, w5, w6 = (W[D * k:D * (k + 1)] for k in range(6))
    b2d = b.reshape(1, D)

    y1 = _tc_matmul(x_values, w1)

    pool1, p0parts, diag = _make_pools_kernel(nnz)(x_values, i0t, i1t, xbs)
    pool0 = _tc_add(p0parts[0], p0parts[1])

    x5 = _make_x5_kernel()(pool0, adg, ast, aps)

    g0, g1 = _tc_node_matmul(diag, pool1, pool0, x5, w2, w3, w4, w5, w6, b2d)

    out = _make_out_kernel(nnz)(g0, g1, y1, i0g, i1g, mpog, mpst, pps)
    return out


# speedup vs baseline: 1.4567x; 1.0387x over previous
import functools

import jax
import jax.numpy as jnp
from jax import lax
from jax.experimental import pallas as pl
from jax.experimental.pallas import tpu as pltpu
from jax.experimental.pallas import tpu_sc as plsc

N = 10000
D = 128
L = 16
NT = 32
PAD = 640

_f32 = jnp.float32
_i32 = jnp.int32


def _wid():
    return lax.axis_index("s") * 2 + lax.axis_index("c")


def _sread(ref, idx):
    return ref[pl.ds(idx, L)][0]


def _clamp_offsets(idxbuf, offbuf, base_val, limit):
    for j in range(128 // L):
        off = idxbuf[pl.ds(L * j, L)] - base_val
        ok = (off >= 0) & (off < limit)
        offbuf[pl.ds(L * j, L)] = jnp.where(ok, off, limit)


def _row_add(dst, dst_row, src, src_row):
    for j in range(D // L):
        v = src[src_row, pl.ds(L * j, L)]
        plsc.addupdate(dst.at[dst_row, pl.ds(L * j, L)], v)


def _zero_rows(buf, nrows):
    z = jnp.zeros((L,), _f32)

    def body(r, _):
        for j in range(D // L):
            buf[r, pl.ds(L * j, L)] = z
        return 0

    lax.fori_loop(0, nrows, body, 0)


def _mm_body(x_ref, w_ref, o_ref):
    o_ref[...] = jnp.dot(x_ref[...], w_ref[...], preferred_element_type=_f32)


def _tc_matmul(x, w):
    nnz = x.shape[0]
    br = 2048
    g = (nnz + br - 1) // br
    return pl.pallas_call(
        _mm_body,
        grid=(g,),
        in_specs=[
            pl.BlockSpec((br, D), lambda i: (i, 0)),
            pl.BlockSpec((D, D), lambda i: (0, 0)),
        ],
        out_specs=pl.BlockSpec((br, D), lambda i: (i, 0)),
        out_shape=jax.ShapeDtypeStruct((nnz, D), _f32),
    )(x, w)


def _node_mm_body(d_ref, p1_ref, p0_ref, x5_ref, w2, w3, w4, w5, w6, b_ref,
                  g0_ref, g1_ref):
    dd = d_ref[...]
    g0_ref[...] = (jnp.dot(dd, w2[...], preferred_element_type=_f32)
                   + jnp.dot(p0_ref[...], w5[...], preferred_element_type=_f32)
                   + jnp.dot(x5_ref[...], w6[...], preferred_element_type=_f32)
                   + b_ref[...])
    g1_ref[...] = (jnp.dot(dd, w3[...], preferred_element_type=_f32)
                   + jnp.dot(p1_ref[...], w4[...], preferred_element_type=_f32))


def _tc_node_matmul(diag, pool1, pool0, x5, w2, w3, w4, w5, w6, b2d):
    br = 1000
    g = N // br
    full = pl.BlockSpec((D, D), lambda i: (0, 0))
    blk = pl.BlockSpec((br, D), lambda i: (i, 0))
    return pl.pallas_call(
        _node_mm_body,
        grid=(g,),
        in_specs=[blk, blk, blk, blk, full, full, full, full, full,
                  pl.BlockSpec((1, D), lambda i: (0, 0))],
        out_specs=[blk, blk],
        out_shape=[jax.ShapeDtypeStruct((N, D), _f32),
                   jax.ShapeDtypeStruct((N, D), _f32)],
    )(diag, pool1, pool0, x5, w2, w3, w4, w5, w6, b2d)


def _make_pools_kernel(nnz):
    mesh = plsc.VectorSubcoreMesh(core_axis_name="c", subcore_axis_name="s")
    SN = 56
    NCH = 6
    NS = 10016
    ZR = 624

    @functools.partial(
        pl.kernel,
        out_type=[jax.ShapeDtypeStruct((N, D), _f32),
                  jax.ShapeDtypeStruct((2, N, D), _f32),
                  jax.ShapeDtypeStruct((N, D), _f32)],
        mesh=mesh,
        scratch_types=[
            pltpu.VMEM((SN + 1, D), _f32),
            pltpu.VMEM((SN + 1, D), _f32),
            pltpu.VMEM((128, D), _f32),
            pltpu.VMEM((128, D), _f32),
            pltpu.VMEM((128,), _i32),
            pltpu.VMEM((128,), _i32),
            pltpu.VMEM((128,), _i32),
            pltpu.VMEM((128,), _i32),
            pltpu.VMEM((128,), _i32),
            pltpu.VMEM((128,), _i32),
            pltpu.VMEM((128,), _i32),
            pltpu.VMEM((128,), _i32),
            pltpu.VMEM((128,), _i32),
            pltpu.VMEM((208,), _i32),
            pltpu.VMEM_SHARED((NS, D), _f32),
            pltpu.SemaphoreType.DMA,
            pltpu.SemaphoreType.DMA,
        ],
    )
    def pools(x_hbm, i0t_hbm, i1t_hbm, xbs_hbm,
              p1_hbm, p0p_hbm, dg_hbm,
              p1buf, dbuf, xbufa, xbufb, i0ba, i0bb, i1ba, i1bb, idba, idbb,
              i1c, offb, dfb, xbsb, spool, sema, semb):
        cid = lax.axis_index("c")
        sid = lax.axis_index("s")
        wid = sid * 2 + cid

        pltpu.sync_copy(xbs_hbm, xbsb)

        _zero_rows(xbufa, 128)
        z0 = pl.multiple_of(sid * ZR, 8)
        for h in range(4):
            pltpu.sync_copy(xbufa.at[pl.ds(0, 128)],
                            spool.at[pl.ds(z0 + 128 * h, 128)])

        @pl.when(sid < 15)
        def _():
            pltpu.sync_copy(xbufa.at[pl.ds(0, 112)],
                            spool.at[pl.ds(z0 + 512, 112)])

        @pl.when(sid == 15)
        def _():
            pltpu.sync_copy(xbufa.at[pl.ds(0, 128)],
                            spool.at[pl.ds(z0 + 512, 128)])

        plsc.subcore_barrier()

        def chunk_body(ci, _):
            n0 = pl.multiple_of(wid * (NCH * SN) + ci * SN, 8)
            r0 = _sread(xbsb, NCH * wid + ci)
            r1 = _sread(xbsb, NCH * wid + ci + 1)
            base = r0 & ~7
            nb = (r1 - base + 127) // 128

            _zero_rows(p1buf, SN + 1)
            _zero_rows(dbuf, SN + 1)

            def issue(bi, i0b, i1b, idb, xbuf, sem):
                st = pl.multiple_of(base + bi * 128, 8)
                pltpu.sync_copy(i0t_hbm.at[pl.ds(st, 128)], i0b)
                pltpu.sync_copy(i1t_hbm.at[pl.ds(st, 128)], i1b)
                for j in range(128 // L):
                    v = st + L * j + lax.iota(_i32, L)
                    idb[pl.ds(L * j, L)] = jnp.minimum(v, nnz - 1)
                pltpu.async_copy(x_hbm.at[idb], xbuf, sem)

            def process(bi, i0b, i1b, xbuf):
                st = pl.multiple_of(base + bi * 128, 8)
                for j in range(128 // L):
                    pos = st + L * j + lax.iota(_i32, L)
                    okp = (pos >= r0) & (pos < r1)
                    i1v = i1b[pl.ds(L * j, L)]
                    i1c[pl.ds(L * j, L)] = jnp.where(okp, i1v, N)
                pltpu.sync_copy(xbuf, spool.at[i1c], add=True)

                _clamp_offsets(i0b, offb, n0, SN)
                for j in range(128 // L):
                    offv = offb[pl.ds(L * j, L)]
                    eq = i0b[pl.ds(L * j, L)] == i1b[pl.ds(L * j, L)]
                    dfb[pl.ds(L * j, L)] = jnp.where(eq, offv, SN)

                def row_body(g, _):
                    gb = g * L
                    offv = offb[pl.ds(gb, L)]
                    dfv = dfb[pl.ds(gb, L)]
                    for u in range(L):
                        _row_add(p1buf, offv[u], xbuf, gb + u)
                        dof = dfv[u]

                        @pl.when(dof < SN)
                        def _():
                            _row_add(dbuf, dof, xbuf, gb + u)

                    return 0

                lax.fori_loop(0, 128 // L, row_body, 0)

            @pl.when(nb > 0)
            def _():
                issue(0, i0ba, i1ba, idba, xbufa, sema)

            def batch_body(bi, _):
                even = (bi % 2) == 0

                @pl.when(even)
                def _():
                    pltpu.make_async_copy(x_hbm.at[idba], xbufa, sema).wait()

                    @pl.when(bi + 1 < nb)
                    def _():
                        issue(bi + 1, i0bb, i1bb, idbb, xbufb, semb)

                    process(bi, i0ba, i1ba, xbufa)

                @pl.when(~even)
                def _():
                    pltpu.make_async_copy(x_hbm.at[idbb], xbufb, semb).wait()

                    @pl.when(bi + 1 < nb)
                    def _():
                        issue(bi + 1, i0ba, i1ba, idba, xbufa, sema)

                    process(bi, i0bb, i1bb, xbufb)

                return 0

            lax.fori_loop(0, nb, batch_body, 0)

            full = n0 + SN <= N
            part = n0 == (N // SN) * SN

            @pl.when(full)
            def _():
                pltpu.sync_copy(p1buf.at[pl.ds(0, SN)], p1_hbm.at[pl.ds(n0, SN)])
                pltpu.sync_copy(dbuf.at[pl.ds(0, SN)], dg_hbm.at[pl.ds(n0, SN)])

            @pl.when(part)
            def _():
                rem = N - (N // SN) * SN
                pltpu.sync_copy(p1buf.at[pl.ds(0, rem)], p1_hbm.at[pl.ds(n0, rem)])
                pltpu.sync_copy(dbuf.at[pl.ds(0, rem)], dg_hbm.at[pl.ds(n0, rem)])

            return 0

        lax.fori_loop(0, NCH, chunk_body, 0)

        plsc.subcore_barrier()
        w0 = pl.multiple_of(sid * ZR, 8)

        @pl.when(sid < 15)
        def _():
            pltpu.sync_copy(spool.at[pl.ds(w0, ZR)],
                            p0p_hbm.at[cid, pl.ds(w0, ZR)])

        @pl.when(sid == 15)
        def _():
            pltpu.sync_copy(spool.at[pl.ds(w0, 640)],
                            p0p_hbm.at[cid, pl.ds(w0, 640)])

    return pools


def _add_body(a_ref, b_ref, o_ref):
    o_ref[...] = a_ref[...] + b_ref[...]


def _tc_add(a, b):
    br = 1000
    blk = pl.BlockSpec((br, D), lambda i: (i, 0))
    return pl.pallas_call(
        _add_body,
        grid=(N // br,),
        in_specs=[blk, blk],
        out_specs=blk,
        out_shape=jax.ShapeDtypeStruct((N, D), _f32),
    )(a, b)


def _make_x5_kernel():
    mesh = plsc.VectorSubcoreMesh(core_axis_name="c", subcore_axis_name="s")
    SN = 320

    @functools.partial(
        pl.kernel,
        out_type=jax.ShapeDtypeStruct((N, D), _f32),
        mesh=mesh,
        scratch_types=[
            pltpu.VMEM((SN + 1, D), _f32),
            pltpu.VMEM((128, D), _f32),
            pltpu.VMEM((128, D), _f32),
            pltpu.VMEM((128,), _i32),
            pltpu.VMEM((128,), _i32),
            pltpu.VMEM((128,), _i32),
            pltpu.VMEM((128,), _i32),
            pltpu.VMEM((128,), _i32),
            pltpu.VMEM((48,), _i32),
            pltpu.SemaphoreType.DMA,
            pltpu.SemaphoreType.DMA,
        ],
    )
    def x5k(p0_hbm, adst_hbm, asrc_hbm, ap_hbm, x5_hbm,
            x5buf, gbufa, gbufb, adba, adbb, asba, asbb, offb, apb,
            sema, semb):
        wid = _wid()
        n0 = pl.multiple_of(wid * SN, 8)
        pltpu.sync_copy(ap_hbm, apb)
        e0 = _sread(apb, wid)
        e1 = _sread(apb, wid + 1)
        base = e0 & ~7
        nb = (e1 - base + 127) // 128

        def issue(bi, adb, asb, gbuf, sem):
            st = pl.multiple_of(base + bi * 128, 8)
            pltpu.sync_copy(adst_hbm.at[pl.ds(st, 128)], adb)
            pltpu.sync_copy(asrc_hbm.at[pl.ds(st, 128)], asb)
            pltpu.async_copy(p0_hbm.at[adb], gbuf, sem)

        def process(asb, gbuf):
            _clamp_offsets(asb, offb, n0, SN)

            def edge_body(g, _):
                gb = g * L
                offv = offb[pl.ds(gb, L)]
                for u in range(L):
                    _row_add(x5buf, offv[u], gbuf, gb + u)
                return 0

            lax.fori_loop(0, 128 // L, edge_body, 0)

        _zero_rows(x5buf, SN + 1)

        @pl.when(nb > 0)
        def _():
            issue(0, adba, asba, gbufa, sema)

        def batch_body(bi, _):
            even = (bi % 2) == 0

            @pl.when(even)
            def _():
                pltpu.make_async_copy(p0_hbm.at[adba], gbufa, sema).wait()

                @pl.when(bi + 1 < nb)
                def _():
                    issue(bi + 1, adbb, asbb, gbufb, semb)

                process(asba, gbufa)

            @pl.when(~even)
            def _():
                pltpu.make_async_copy(p0_hbm.at[adbb], gbufb, semb).wait()

                @pl.when(bi + 1 < nb)
                def _():
                    issue(bi + 1, adba, asba, gbufa, sema)

                process(asbb, gbufb)

            return 0

        lax.fori_loop(0, nb, batch_body, 0)

        full = n0 + SN <= N
        part = n0 == (N // SN) * SN

        @pl.when(full)
        def _():
            pltpu.sync_copy(x5buf.at[pl.ds(0, SN)], x5_hbm.at[pl.ds(n0, SN)])

        @pl.when(part)
        def _():
            rem = N - (N // SN) * SN
            pltpu.sync_copy(x5buf.at[pl.ds(0, rem)], x5_hbm.at[pl.ds(n0, rem)])

    return x5k


def _make_out_kernel(nnz):
    mesh = plsc.VectorSubcoreMesh(core_axis_name="c", subcore_axis_name="s")
    SR = 256
    nch_total = (nnz + SR - 1) // SR
    last_c = nch_total - 1
    lastv = nnz - last_c * SR
    base_nch = nch_total // NT
    extra = nch_total - base_nch * NT

    @functools.partial(
        pl.kernel,
        out_type=jax.ShapeDtypeStruct((nnz, D), _f32),
        mesh=mesh,
        scratch_types=[
            pltpu.VMEM((SR + 1, D), _f32),
            pltpu.VMEM((SR, D), _f32),
            pltpu.VMEM((128, D), _f32),
            pltpu.VMEM((128, D), _f32),
            pltpu.VMEM((128,), _i32),
            pltpu.VMEM((128,), _i32),
            pltpu.VMEM((128,), _i32),
            pltpu.VMEM((128,), _i32),
            pltpu.VMEM((128,), _i32),
            pltpu.VMEM((128,), _i32),
            pltpu.VMEM((128,), _i32),
            pltpu.VMEM((128,), _i32),
            pltpu.VMEM((128,), _i32),
            pltpu.VMEM((688,), _i32),
            pltpu.SemaphoreType.DMA,
            pltpu.SemaphoreType.DMA,
            pltpu.SemaphoreType.DMA,
            pltpu.SemaphoreType.DMA,
            pltpu.SemaphoreType.DMA,
            pltpu.SemaphoreType.DMA,
        ],
    )
    def outk(g0_hbm, g1_hbm, y1_hbm, i0_hbm, i1_hbm, mpo_hbm, mps_hbm, pp_hbm,
             out_hbm, outbuf, bbuf, ybufa, ybufb, ib0, ib1, ib2, ib3,
             poba, pobb, psba, psbb, offb, ppb, sem0, sem1, sem2, sem3,
             sema, semb):
        wid = _wid()
        nch = base_nch + jnp.where(wid < extra, 1, 0)
        pltpu.sync_copy(pp_hbm, ppb)

        def issue(base, bi, pob, psb, ybuf, sem):
            st = pl.multiple_of(base + bi * 128, 8)
            pltpu.sync_copy(mpo_hbm.at[pl.ds(st, 128)], pob)
            pltpu.sync_copy(mps_hbm.at[pl.ds(st, 128)], psb)
            pltpu.async_copy(y1_hbm.at[pob], ybuf, sem)

        def process(ar, psb, ybuf):
            _clamp_offsets(psb, offb, ar, SR)

            def pair_body(g, _):
                gb = g * L
                offv = offb[pl.ds(gb, L)]
                for u in range(L):
                    _row_add(outbuf, offv[u], ybuf, gb + u)
                return 0

            lax.fori_loop(0, 128 // L, pair_body, 0)

        def chunk_body(ci, _):
            c = wid + NT * ci
            ar = pl.multiple_of(c * SR, 8)
            p0 = _sread(ppb, c)
            p1 = _sread(ppb, c + 1)
            base = p0 & ~7
            nb = (p1 - base + 127) // 128

            pltpu.sync_copy(i0_hbm.at[pl.ds(ar, 128)], ib0)
            pltpu.sync_copy(i0_hbm.at[pl.ds(ar + 128, 128)], ib1)
            pltpu.sync_copy(i1_hbm.at[pl.ds(ar, 128)], ib2)
            pltpu.sync_copy(i1_hbm.at[pl.ds(ar + 128, 128)], ib3)
            c0 = pltpu.async_copy(g0_hbm.at[ib0], outbuf.at[pl.ds(0, 128)], sem0)
            c1 = pltpu.async_copy(g0_hbm.at[ib1], outbuf.at[pl.ds(128, 128)], sem1)
            c2 = pltpu.async_copy(g1_hbm.at[ib2], bbuf.at[pl.ds(0, 128)], sem2)
            c3 = pltpu.async_copy(g1_hbm.at[ib3], bbuf.at[pl.ds(128, 128)], sem3)

            @pl.when(nb > 0)
            def _():
                issue(base, 0, poba, psba, ybufa, sema)

            c0.wait()
            c1.wait()
            c2.wait()
            c3.wait()

            def init_body(g, _):
                for u in range(2):
                    r = 2 * g + u
                    _row_add(outbuf, r, bbuf, r)
                return 0

            lax.fori_loop(0, SR // 2, init_body, 0)

            def batch_body(bi, _):
                even = (bi % 2) == 0

                @pl.when(even)
                def _():
                    pltpu.make_async_copy(y1_hbm.at[poba], ybufa, sema).wait()

                    @pl.when(bi + 1 < nb)
                    def _():
                        issue(base, bi + 1, pobb, psbb, ybufb, semb)

                    process(ar, psba, ybufa)

                @pl.when(~even)
                def _():
                    pltpu.make_async_copy(y1_hbm.at[pobb], ybufb, semb).wait()

                    @pl.when(bi + 1 < nb)
                    def _():
                        issue(base, bi + 1, poba, psba, ybufa, sema)

                    process(ar, psbb, ybufb)

                return 0

            lax.fori_loop(0, nb, batch_body, 0)

            @pl.when(c != last_c)
            def _():
                pltpu.sync_copy(outbuf.at[pl.ds(0, SR)], out_hbm.at[pl.ds(ar, SR)])

            @pl.when(c == last_c)
            def _():
                pltpu.sync_copy(outbuf.at[pl.ds(0, lastv)],
                                out_hbm.at[pl.ds(last_c * SR, lastv)])

            return 0

        lax.fori_loop(0, nch, chunk_body, 0)

    return outk


def kernel(x_values, W, b, x_indices, a_indices, mp_src, mp_out):
    nnz = x_values.shape[0]
    i0 = x_indices[0].astype(_i32)
    i1 = x_indices[1].astype(_i32)
    a_src = a_indices[0].astype(_i32)
    a_dst = a_indices[1].astype(_i32)
    mps = mp_src.astype(_i32)
    mpo = mp_out.astype(_i32)

    SRK4 = 256
    nch = (nnz + SRK4 - 1) // SRK4
    pps = jnp.searchsorted(mps, jnp.arange(0, (nch + 1) * SRK4, SRK4,
                                           dtype=_i32)).astype(_i32)
    pps = jnp.pad(pps, (0, 688 - pps.shape[0]))
    xbs = jnp.searchsorted(i0, jnp.arange(0, 10753, 56, dtype=_i32)).astype(_i32)
    xbs = jnp.pad(xbs, (0, 208 - xbs.shape[0]), constant_values=nnz)
    aps = jnp.searchsorted(a_src, jnp.arange(0, 10241, 320, dtype=_i32)).astype(_i32)
    aps = jnp.pad(aps, (0, 48 - aps.shape[0]), constant_values=a_src.shape[0])

    i0t = jnp.pad(i0, (0, PAD), constant_values=-1)
    i1t = jnp.pad(i1, (0, PAD), constant_values=-2)
    i0g = jnp.pad(i0, (0, PAD), constant_values=0)
    i1g = jnp.pad(i1, (0, PAD), constant_values=0)
    mpog = jnp.pad(mpo, (0, PAD), constant_values=0)
    mpst = jnp.pad(mps, (0, PAD), constant_values=-1)
    adg = jnp.pad(a_dst, (0, PAD), constant_values=0)
    ast = jnp.pad(a_src, (0, PAD), constant_values=-1)

    w1, w2, w3, w4, w5, w6 = (W[D * k:D * (k + 1)] for k in range(6))
    b2d = b.reshape(1, D)

    y1 = _tc_matmul(x_values, w1)

    pool1, p0parts, diag = _make_pools_kernel(nnz)(x_values, i0t, i1t, xbs)
    pool0 = _tc_add(p0parts[0], p0parts[1])

    x5 = _make_x5_kernel()(pool0, adg, ast, aps)

    g0, g1 = _tc_node_matmul(diag, pool1, pool0, x5, w2, w3, w4, w5, w6, b2d)

    out = _make_out_kernel(nnz)(g0, g1, y1, i0g, i1g, mpog, mpst, pps)
    return out


# speedup vs baseline: 1.4944x; 1.0259x over previous
import functools

import jax
import jax.numpy as jnp
from jax import lax
from jax.experimental import pallas as pl
from jax.experimental.pallas import tpu as pltpu
from jax.experimental.pallas import tpu_sc as plsc

N = 10000
D = 128
L = 16
NT = 32
PAD = 640

_f32 = jnp.float32
_i32 = jnp.int32


def _wid():
    return lax.axis_index("s") * 2 + lax.axis_index("c")


def _sread(ref, idx):
    return ref[pl.ds(idx, L)][0]


def _clamp_offsets(idxbuf, offbuf, base_val, limit):
    for j in range(128 // L):
        off = idxbuf[pl.ds(L * j, L)] - base_val
        ok = (off >= 0) & (off < limit)
        offbuf[pl.ds(L * j, L)] = jnp.where(ok, off, limit)


def _row_add(dst, dst_row, src, src_row):
    for j in range(D // L):
        v = src[src_row, pl.ds(L * j, L)]
        plsc.addupdate(dst.at[dst_row, pl.ds(L * j, L)], v)


def _zero_rows(buf, nrows):
    z = jnp.zeros((L,), _f32)

    def body(r, _):
        for j in range(D // L):
            buf[r, pl.ds(L * j, L)] = z
        return 0

    lax.fori_loop(0, nrows, body, 0)


def _mm_body(x_ref, w_ref, o_ref):
    o_ref[...] = jnp.dot(x_ref[...], w_ref[...], preferred_element_type=_f32)


def _tc_matmul(x, w):
    nnz = x.shape[0]
    br = 2048
    g = (nnz + br - 1) // br
    return pl.pallas_call(
        _mm_body,
        grid=(g,),
        in_specs=[
            pl.BlockSpec((br, D), lambda i: (i, 0)),
            pl.BlockSpec((D, D), lambda i: (0, 0)),
        ],
        out_specs=pl.BlockSpec((br, D), lambda i: (i, 0)),
        out_shape=jax.ShapeDtypeStruct((nnz, D), _f32),
    )(x, w)


def _node_mm_body(d_ref, p1_ref, p0_ref, x5_ref, w2, w3, w4, w5, w6, b_ref,
                  g0_ref, g1_ref):
    dd = d_ref[...]
    g0_ref[...] = (jnp.dot(dd, w2[...], preferred_element_type=_f32)
                   + jnp.dot(p0_ref[...], w5[...], preferred_element_type=_f32)
                   + jnp.dot(x5_ref[...], w6[...], preferred_element_type=_f32)
                   + b_ref[...])
    g1_ref[...] = (jnp.dot(dd, w3[...], preferred_element_type=_f32)
                   + jnp.dot(p1_ref[...], w4[...], preferred_element_type=_f32))


def _tc_node_matmul(diag, pool1, pool0, x5, w2, w3, w4, w5, w6, b2d):
    br = 1000
    g = N // br
    full = pl.BlockSpec((D, D), lambda i: (0, 0))
    blk = pl.BlockSpec((br, D), lambda i: (i, 0))
    return pl.pallas_call(
        _node_mm_body,
        grid=(g,),
        in_specs=[blk, blk, blk, blk, full, full, full, full, full,
                  pl.BlockSpec((1, D), lambda i: (0, 0))],
        out_specs=[blk, blk],
        out_shape=[jax.ShapeDtypeStruct((N, D), _f32),
                   jax.ShapeDtypeStruct((N, D), _f32)],
    )(diag, pool1, pool0, x5, w2, w3, w4, w5, w6, b2d)


def _make_pools_kernel(nnz):
    mesh = plsc.VectorSubcoreMesh(core_axis_name="c", subcore_axis_name="s")
    SN = 56
    NCH = 6
    NS = 10016
    ZR = 624

    @functools.partial(
        pl.kernel,
        out_type=[jax.ShapeDtypeStruct((N, D), _f32),
                  jax.ShapeDtypeStruct((2, N, D), _f32),
                  jax.ShapeDtypeStruct((N, D), _f32)],
        mesh=mesh,
        scratch_types=[
            pltpu.VMEM((SN + 1, D), _f32),
            pltpu.VMEM((SN + 1, D), _f32),
            pltpu.VMEM((128, D), _f32),
            pltpu.VMEM((128, D), _f32),
            pltpu.VMEM((128,), _i32),
            pltpu.VMEM((128,), _i32),
            pltpu.VMEM((128,), _i32),
            pltpu.VMEM((128,), _i32),
            pltpu.VMEM((128,), _i32),
            pltpu.VMEM((128,), _i32),
            pltpu.VMEM((128,), _i32),
            pltpu.VMEM((128,), _i32),
            pltpu.VMEM((128,), _i32),
            pltpu.VMEM((128,), _i32),
            pltpu.VMEM((208,), _i32),
            pltpu.VMEM_SHARED((NS, D), _f32),
            pltpu.SemaphoreType.DMA,
            pltpu.SemaphoreType.DMA,
            pltpu.SemaphoreType.DMA,
            pltpu.SemaphoreType.DMA,
        ],
    )
    def pools(x_hbm, i0t_hbm, i1t_hbm, xbs_hbm,
              p1_hbm, p0p_hbm, dg_hbm,
              p1buf, dbuf, xbufa, xbufb, i0ba, i0bb, i1ba, i1bb, idba, idbb,
              i1ca, i1cb, offb, dfb, xbsb, spool, sema, semb, semsa, semsb):
        cid = lax.axis_index("c")
        sid = lax.axis_index("s")
        wid = sid * 2 + cid

        pltpu.sync_copy(xbs_hbm, xbsb)

        _zero_rows(xbufa, 128)
        z0 = pl.multiple_of(sid * ZR, 8)
        for h in range(4):
            pltpu.sync_copy(xbufa.at[pl.ds(0, 128)],
                            spool.at[pl.ds(z0 + 128 * h, 128)])

        @pl.when(sid < 15)
        def _():
            pltpu.sync_copy(xbufa.at[pl.ds(0, 112)],
                            spool.at[pl.ds(z0 + 512, 112)])

        @pl.when(sid == 15)
        def _():
            pltpu.sync_copy(xbufa.at[pl.ds(0, 128)],
                            spool.at[pl.ds(z0 + 512, 128)])

        plsc.subcore_barrier()

        def chunk_body(ci, _):
            n0 = pl.multiple_of(wid * (NCH * SN) + ci * SN, 8)
            r0 = _sread(xbsb, NCH * wid + ci)
            r1 = _sread(xbsb, NCH * wid + ci + 1)
            base = r0 & ~7
            nb = (r1 - base + 127) // 128

            _zero_rows(p1buf, SN + 1)
            _zero_rows(dbuf, SN + 1)

            def issue(bi, i0b, i1b, idb, xbuf, sem, i1c, sems):
                @pl.when(bi >= 2)
                def _():
                    pltpu.make_async_copy(xbuf, spool.at[i1c], sems).wait()

                st = pl.multiple_of(base + bi * 128, 8)
                pltpu.sync_copy(i0t_hbm.at[pl.ds(st, 128)], i0b)
                pltpu.sync_copy(i1t_hbm.at[pl.ds(st, 128)], i1b)
                for j in range(128 // L):
                    v = st + L * j + lax.iota(_i32, L)
                    idb[pl.ds(L * j, L)] = jnp.minimum(v, nnz - 1)
                pltpu.async_copy(x_hbm.at[idb], xbuf, sem)

            def process(bi, i0b, i1b, xbuf, i1c, sems):
                st = pl.multiple_of(base + bi * 128, 8)
                for j in range(128 // L):
                    pos = st + L * j + lax.iota(_i32, L)
                    okp = (pos >= r0) & (pos < r1)
                    i1v = i1b[pl.ds(L * j, L)]
                    i1c[pl.ds(L * j, L)] = jnp.where(okp, i1v, N)
                pltpu.async_copy(xbuf, spool.at[i1c], sems, add=True)

                _clamp_offsets(i0b, offb, n0, SN)
                for j in range(128 // L):
                    offv = offb[pl.ds(L * j, L)]
                    eq = i0b[pl.ds(L * j, L)] == i1b[pl.ds(L * j, L)]
                    dfb[pl.ds(L * j, L)] = jnp.where(eq, offv, SN)

                def row_body(g, _):
                    gb = g * L
                    offv = offb[pl.ds(gb, L)]
                    dfv = dfb[pl.ds(gb, L)]
                    offs = [offv[u] for u in range(L)]
                    dofs = [dfv[u] for u in range(L)]
                    for u in range(L):
                        _row_add(p1buf, offs[u], xbuf, gb + u)

                        @pl.when(dofs[u] < SN)
                        def _():
                            _row_add(dbuf, dofs[u], xbuf, gb + u)

                    return 0

                lax.fori_loop(0, 128 // L, row_body, 0)

            @pl.when(nb > 0)
            def _():
                issue(0, i0ba, i1ba, idba, xbufa, sema, i1ca, semsa)

            def batch_body(bi, _):
                even = (bi % 2) == 0

                @pl.when(even)
                def _():
                    pltpu.make_async_copy(x_hbm.at[idba], xbufa, sema).wait()

                    @pl.when(bi + 1 < nb)
                    def _():
                        issue(bi + 1, i0bb, i1bb, idbb, xbufb, semb, i1cb, semsb)

                    process(bi, i0ba, i1ba, xbufa, i1ca, semsa)

                @pl.when(~even)
                def _():
                    pltpu.make_async_copy(x_hbm.at[idbb], xbufb, semb).wait()

                    @pl.when(bi + 1 < nb)
                    def _():
                        issue(bi + 1, i0ba, i1ba, idba, xbufa, sema, i1ca, semsa)

                    process(bi, i0bb, i1bb, xbufb, i1cb, semsb)

                return 0

            lax.fori_loop(0, nb, batch_body, 0)

            @pl.when(nb >= 2)
            def _():
                lastm1 = ((nb - 2) % 2) == 0

                @pl.when(lastm1)
                def _():
                    pltpu.make_async_copy(xbufa, spool.at[i1ca], semsa).wait()

                @pl.when(~lastm1)
                def _():
                    pltpu.make_async_copy(xbufb, spool.at[i1cb], semsb).wait()

            @pl.when(nb >= 1)
            def _():
                lastp = ((nb - 1) % 2) == 0

                @pl.when(lastp)
                def _():
                    pltpu.make_async_copy(xbufa, spool.at[i1ca], semsa).wait()

                @pl.when(~lastp)
                def _():
                    pltpu.make_async_copy(xbufb, spool.at[i1cb], semsb).wait()

            full = n0 + SN <= N
            part = n0 == (N // SN) * SN

            @pl.when(full)
            def _():
                pltpu.sync_copy(p1buf.at[pl.ds(0, SN)], p1_hbm.at[pl.ds(n0, SN)])
                pltpu.sync_copy(dbuf.at[pl.ds(0, SN)], dg_hbm.at[pl.ds(n0, SN)])

            @pl.when(part)
            def _():
                rem = N - (N // SN) * SN
                pltpu.sync_copy(p1buf.at[pl.ds(0, rem)], p1_hbm.at[pl.ds(n0, rem)])
                pltpu.sync_copy(dbuf.at[pl.ds(0, rem)], dg_hbm.at[pl.ds(n0, rem)])

            return 0

        lax.fori_loop(0, NCH, chunk_body, 0)

        plsc.subcore_barrier()
        w0 = pl.multiple_of(sid * ZR, 8)

        @pl.when(sid < 15)
        def _():
            pltpu.sync_copy(spool.at[pl.ds(w0, ZR)],
                            p0p_hbm.at[cid, pl.ds(w0, ZR)])

        @pl.when(sid == 15)
        def _():
            pltpu.sync_copy(spool.at[pl.ds(w0, 640)],
                            p0p_hbm.at[cid, pl.ds(w0, 640)])

    return pools


def _add_body(a_ref, b_ref, o_ref):
    o_ref[...] = a_ref[...] + b_ref[...]


def _tc_add(a, b):
    br = 1000
    blk = pl.BlockSpec((br, D), lambda i: (i, 0))
    return pl.pallas_call(
        _add_body,
        grid=(N // br,),
        in_specs=[blk, blk],
        out_specs=blk,
        out_shape=jax.ShapeDtypeStruct((N, D), _f32),
    )(a, b)


def _make_x5_kernel():
    mesh = plsc.VectorSubcoreMesh(core_axis_name="c", subcore_axis_name="s")
    SN = 320

    @functools.partial(
        pl.kernel,
        out_type=jax.ShapeDtypeStruct((N, D), _f32),
        mesh=mesh,
        scratch_types=[
            pltpu.VMEM((SN + 1, D), _f32),
            pltpu.VMEM((128, D), _f32),
            pltpu.VMEM((128, D), _f32),
            pltpu.VMEM((128,), _i32),
            pltpu.VMEM((128,), _i32),
            pltpu.VMEM((128,), _i32),
            pltpu.VMEM((128,), _i32),
            pltpu.VMEM((128,), _i32),
            pltpu.VMEM((48,), _i32),
            pltpu.SemaphoreType.DMA,
            pltpu.SemaphoreType.DMA,
        ],
    )
    def x5k(p0_hbm, adst_hbm, asrc_hbm, ap_hbm, x5_hbm,
            x5buf, gbufa, gbufb, adba, adbb, asba, asbb, offb, apb,
            sema, semb):
        wid = _wid()
        n0 = pl.multiple_of(wid * SN, 8)
        pltpu.sync_copy(ap_hbm, apb)
        e0 = _sread(apb, wid)
        e1 = _sread(apb, wid + 1)
        base = e0 & ~7
        nb = (e1 - base + 127) // 128

        def issue(bi, adb, asb, gbuf, sem):
            st = pl.multiple_of(base + bi * 128, 8)
            pltpu.sync_copy(adst_hbm.at[pl.ds(st, 128)], adb)
            pltpu.sync_copy(asrc_hbm.at[pl.ds(st, 128)], asb)
            pltpu.async_copy(p0_hbm.at[adb], gbuf, sem)

        def process(asb, gbuf):
            _clamp_offsets(asb, offb, n0, SN)

            def edge_body(g, _):
                gb = g * L
                offv = offb[pl.ds(gb, L)]
                offs = [offv[u] for u in range(L)]
                for u in range(L):
                    _row_add(x5buf, offs[u], gbuf, gb + u)
                return 0

            lax.fori_loop(0, 128 // L, edge_body, 0)

        _zero_rows(x5buf, SN + 1)

        @pl.when(nb > 0)
        def _():
            issue(0, adba, asba, gbufa, sema)

        def batch_body(bi, _):
            even = (bi % 2) == 0

            @pl.when(even)
            def _():
                pltpu.make_async_copy(p0_hbm.at[adba], gbufa, sema).wait()

                @pl.when(bi + 1 < nb)
                def _():
                    issue(bi + 1, adbb, asbb, gbufb, semb)

                process(asba, gbufa)

            @pl.when(~even)
            def _():
                pltpu.make_async_copy(p0_hbm.at[adbb], gbufb, semb).wait()

                @pl.when(bi + 1 < nb)
                def _():
                    issue(bi + 1, adba, asba, gbufa, sema)

                process(asbb, gbufb)

            return 0

        lax.fori_loop(0, nb, batch_body, 0)

        full = n0 + SN <= N
        part = n0 == (N // SN) * SN

        @pl.when(full)
        def _():
            pltpu.sync_copy(x5buf.at[pl.ds(0, SN)], x5_hbm.at[pl.ds(n0, SN)])

        @pl.when(part)
        def _():
            rem = N - (N // SN) * SN
            pltpu.sync_copy(x5buf.at[pl.ds(0, rem)], x5_hbm.at[pl.ds(n0, rem)])

    return x5k


def _make_out_kernel(nnz):
    mesh = plsc.VectorSubcoreMesh(core_axis_name="c", subcore_axis_name="s")
    SR = 256
    nch_total = (nnz + SR - 1) // SR
    last_c = nch_total - 1
    lastv = nnz - last_c * SR
    base_nch = nch_total // NT
    extra = nch_total - base_nch * NT

    @functools.partial(
        pl.kernel,
        out_type=jax.ShapeDtypeStruct((nnz, D), _f32),
        mesh=mesh,
        scratch_types=[
            pltpu.VMEM((SR + 1, D), _f32),
            pltpu.VMEM((SR, D), _f32),
            pltpu.VMEM((128, D), _f32),
            pltpu.VMEM((128, D), _f32),
            pltpu.VMEM((128,), _i32),
            pltpu.VMEM((128,), _i32),
            pltpu.VMEM((128,), _i32),
            pltpu.VMEM((128,), _i32),
            pltpu.VMEM((128,), _i32),
            pltpu.VMEM((128,), _i32),
            pltpu.VMEM((128,), _i32),
            pltpu.VMEM((128,), _i32),
            pltpu.VMEM((128,), _i32),
            pltpu.VMEM((688,), _i32),
            pltpu.SemaphoreType.DMA,
            pltpu.SemaphoreType.DMA,
            pltpu.SemaphoreType.DMA,
            pltpu.SemaphoreType.DMA,
            pltpu.SemaphoreType.DMA,
            pltpu.SemaphoreType.DMA,
        ],
    )
    def outk(g0_hbm, g1_hbm, y1_hbm, i0_hbm, i1_hbm, mpo_hbm, mps_hbm, pp_hbm,
             out_hbm, outbuf, bbuf, ybufa, ybufb, ib0, ib1, ib2, ib3,
             poba, pobb, psba, psbb, offb, ppb, sem0, sem1, sem2, sem3,
             sema, semb):
        wid = _wid()
        nch = base_nch + jnp.where(wid < extra, 1, 0)
        pltpu.sync_copy(pp_hbm, ppb)

        def issue(base, bi, pob, psb, ybuf, sem):
            st = pl.multiple_of(base + bi * 128, 8)
            pltpu.sync_copy(mpo_hbm.at[pl.ds(st, 128)], pob)
            pltpu.sync_copy(mps_hbm.at[pl.ds(st, 128)], psb)
            pltpu.async_copy(y1_hbm.at[pob], ybuf, sem)

        def process(ar, psb, ybuf):
            _clamp_offsets(psb, offb, ar, SR)

            def pair_body(g, _):
                gb = g * L
                offv = offb[pl.ds(gb, L)]
                offs = [offv[u] for u in range(L)]
                for u in range(L):
                    _row_add(outbuf, offs[u], ybuf, gb + u)
                return 0

            lax.fori_loop(0, 128 // L, pair_body, 0)

        def chunk_body(ci, _):
            c = wid + NT * ci
            ar = pl.multiple_of(c * SR, 8)
            p0 = _sread(ppb, c)
            p1 = _sread(ppb, c + 1)
            base = p0 & ~7
            nb = (p1 - base + 127) // 128

            pltpu.sync_copy(i0_hbm.at[pl.ds(ar, 128)], ib0)
            pltpu.sync_copy(i0_hbm.at[pl.ds(ar + 128, 128)], ib1)
            pltpu.sync_copy(i1_hbm.at[pl.ds(ar, 128)], ib2)
            pltpu.sync_copy(i1_hbm.at[pl.ds(ar + 128, 128)], ib3)
            c0 = pltpu.async_copy(g0_hbm.at[ib0], outbuf.at[pl.ds(0, 128)], sem0)
            c1 = pltpu.async_copy(g0_hbm.at[ib1], outbuf.at[pl.ds(128, 128)], sem1)
            c2 = pltpu.async_copy(g1_hbm.at[ib2], bbuf.at[pl.ds(0, 128)], sem2)
            c3 = pltpu.async_copy(g1_hbm.at[ib3], bbuf.at[pl.ds(128, 128)], sem3)

            @pl.when(nb > 0)
            def _():
                issue(base, 0, poba, psba, ybufa, sema)

            c0.wait()
            c1.wait()
            c2.wait()
            c3.wait()

            def init_body(g, _):
                for u in range(2):
                    r = 2 * g + u
                    _row_add(outbuf, r, bbuf, r)
                return 0

            lax.fori_loop(0, SR // 2, init_body, 0)

            def batch_body(bi, _):
                even = (bi % 2) == 0

                @pl.when(even)
                def _():
                    pltpu.make_async_copy(y1_hbm.at[poba], ybufa, sema).wait()

                    @pl.when(bi + 1 < nb)
                    def _():
                        issue(base, bi + 1, pobb, psbb, ybufb, semb)

                    process(ar, psba, ybufa)

                @pl.when(~even)
                def _():
                    pltpu.make_async_copy(y1_hbm.at[pobb], ybufb, semb).wait()

                    @pl.when(bi + 1 < nb)
                    def _():
                        issue(base, bi + 1, poba, psba, ybufa, sema)

                    process(ar, psbb, ybufb)

                return 0

            lax.fori_loop(0, nb, batch_body, 0)

            @pl.when(c != last_c)
            def _():
                pltpu.sync_copy(outbuf.at[pl.ds(0, SR)], out_hbm.at[pl.ds(ar, SR)])

            @pl.when(c == last_c)
            def _():
                pltpu.sync_copy(outbuf.at[pl.ds(0, lastv)],
                                out_hbm.at[pl.ds(last_c * SR, lastv)])

            return 0

        lax.fori_loop(0, nch, chunk_body, 0)

    return outk


def kernel(x_values, W, b, x_indices, a_indices, mp_src, mp_out):
    nnz = x_values.shape[0]
    i0 = x_indices[0].astype(_i32)
    i1 = x_indices[1].astype(_i32)
    a_src = a_indices[0].astype(_i32)
    a_dst = a_indices[1].astype(_i32)
    mps = mp_src.astype(_i32)
    mpo = mp_out.astype(_i32)

    SRK4 = 256
    nch = (nnz + SRK4 - 1) // SRK4
    pps = jnp.searchsorted(mps, jnp.arange(0, (nch + 1) * SRK4, SRK4,
                                           dtype=_i32)).astype(_i32)
    pps = jnp.pad(pps, (0, 688 - pps.shape[0]))
    xbs = jnp.searchsorted(i0, jnp.arange(0, 10753, 56, dtype=_i32)).astype(_i32)
    xbs = jnp.pad(xbs, (0, 208 - xbs.shape[0]), constant_values=nnz)
    aps = jnp.searchsorted(a_src, jnp.arange(0, 10241, 320, dtype=_i32)).astype(_i32)
    aps = jnp.pad(aps, (0, 48 - aps.shape[0]), constant_values=a_src.shape[0])

    i0t = jnp.pad(i0, (0, PAD), constant_values=-1)
    i1t = jnp.pad(i1, (0, PAD), constant_values=-2)
    i0g = jnp.pad(i0, (0, PAD), constant_values=0)
    i1g = jnp.pad(i1, (0, PAD), constant_values=0)
    mpog = jnp.pad(mpo, (0, PAD), constant_values=0)
    mpst = jnp.pad(mps, (0, PAD), constant_values=-1)
    adg = jnp.pad(a_dst, (0, PAD), constant_values=0)
    ast = jnp.pad(a_src, (0, PAD), constant_values=-1)

    w1, w2, w3, w4, w5, w6 = (W[D * k:D * (k + 1)] for k in range(6))
    b2d = b.reshape(1, D)

    y1 = _tc_matmul(x_values, w1)

    pool1, p0parts, diag = _make_pools_kernel(nnz)(x_values, i0t, i1t, xbs)
    pool0 = _tc_add(p0parts[0], p0parts[1])

    x5 = _make_x5_kernel()(pool0, adg, ast, aps)

    g0, g1 = _tc_node_matmul(diag, pool1, pool0, x5, w2, w3, w4, w5, w6, b2d)

    out = _make_out_kernel(nnz)(g0, g1, y1, i0g, i1g, mpog, mpst, pps)
    return out
